# Initial kernel scaffold; baseline (speedup 1.0000x reference)
import functools

import jax
import jax.numpy as jnp
from jax import lax
from jax.experimental import pallas as pl
from jax.experimental.pallas import tpu as pltpu
from jax.experimental.pallas import tpu_sc as plsc

N = 10000
E = 640000
H = 4
DH = 32
F = H * DH
EXT = F + 16
NEG = 0.2

HALF = N // 2
DUMMY = HALF
OUTS_ROWS = 5120
NT = 16
EPT = E // NT
CHUNK = 800
NCHUNK = EPT // CHUNK
SB = 128
CCAP = 1024
FROWS = N // NT
STRIPE = OUTS_ROWS // NT
OROWS = 313


_BLK = 1250


def _feat_body(h_ref, W_ref, Al_ref, Ar_ref, feat_ref, elr_ref):
    feat = jnp.dot(h_ref[...], W_ref[...], preferred_element_type=jnp.float32)
    feat_ref[...] = feat
    el = jnp.dot(feat, Al_ref[...], preferred_element_type=jnp.float32)
    er = jnp.dot(feat, Ar_ref[...], preferred_element_type=jnp.float32)
    elr_ref[...] = jnp.concatenate([el, er], axis=1)


def _tc_feat(h, W, Al, Ar):
    return pl.pallas_call(
        _feat_body,
        grid=(N // _BLK,),
        in_specs=[
            pl.BlockSpec((_BLK, F), lambda i: (i, 0)),
            pl.BlockSpec((F, F), lambda i: (0, 0)),
            pl.BlockSpec((F, H), lambda i: (0, 0)),
            pl.BlockSpec((F, H), lambda i: (0, 0)),
        ],
        out_specs=[
            pl.BlockSpec((_BLK, F), lambda i: (i, 0)),
            pl.BlockSpec((_BLK, 2 * H), lambda i: (i, 0)),
        ],
        out_shape=[
            jax.ShapeDtypeStruct((N, F), jnp.float32),
            jax.ShapeDtypeStruct((N, 2 * H), jnp.float32),
        ],
    )(h, W, Al, Ar)


def _normalize(oe):
    cols = []
    for hh in range(H):
        s = oe[:, F + hh:F + hh + 1] + 1e-9
        cols.append(oe[:, hh * DH:(hh + 1) * DH] / s)
    return jnp.concatenate(cols, axis=1)


def _post_feat_body(residual, act, oe_ref, hp_ref, b_ref, W_ref, Al_ref, Ar_ref,
                    h_ref, feat_ref, elr_ref):
    rst = _normalize(oe_ref[...])
    if residual:
        rst = rst + hp_ref[...]
    rst = rst + b_ref[...]
    if act:
        rst = jnp.where(rst > 0, rst, jnp.exp(jnp.minimum(rst, 0.0)) - 1.0)
    h_ref[...] = rst
    feat = jnp.dot(rst, W_ref[...], preferred_element_type=jnp.float32)
    feat_ref[...] = feat
    el = jnp.dot(feat, Al_ref[...], preferred_element_type=jnp.float32)
    er = jnp.dot(feat, Ar_ref[...], preferred_element_type=jnp.float32)
    elr_ref[...] = jnp.concatenate([el, er], axis=1)


def _tc_post_feat(oe, hp, b2, W, Al, Ar, residual, act):
    return pl.pallas_call(
        functools.partial(_post_feat_body, residual, act),
        grid=(N // _BLK,),
        in_specs=[
            pl.BlockSpec((_BLK, EXT), lambda i: (i, 0)),
            pl.BlockSpec((_BLK, F), lambda i: (i, 0)),
            pl.BlockSpec((1, F), lambda i: (0, 0)),
            pl.BlockSpec((F, F), lambda i: (0, 0)),
            pl.BlockSpec((F, H), lambda i: (0, 0)),
            pl.BlockSpec((F, H), lambda i: (0, 0)),
        ],
        out_specs=[
            pl.BlockSpec((_BLK, F), lambda i: (i, 0)),
            pl.BlockSpec((_BLK, F), lambda i: (i, 0)),
            pl.BlockSpec((_BLK, 2 * H), lambda i: (i, 0)),
        ],
        out_shape=[
            jax.ShapeDtypeStruct((N, F), jnp.float32),
            jax.ShapeDtypeStruct((N, F), jnp.float32),
            jax.ShapeDtypeStruct((N, 2 * H), jnp.float32),
        ],
    )(oe, hp, b2, W, Al, Ar)


def _final_body(oe_ref, hp_ref, b_ref, Wp_ref, bp_ref, Wv_ref, bv_ref,
                pi_ref, v_ref, acc_ref):
    i = pl.program_id(0)
    rst = _normalize(oe_ref[...]) + hp_ref[...] + b_ref[...]
    hm = (rst[:, 0:DH] + rst[:, DH:2 * DH] + rst[:, 2 * DH:3 * DH]
          + rst[:, 3 * DH:4 * DH]) * 0.25
    pi_ref[...] = (jnp.dot(hm, Wp_ref[...], preferred_element_type=jnp.float32)
                   + bp_ref[...])

    @pl.when(i == 0)
    def _():
        acc_ref[...] = jnp.zeros_like(acc_ref)

    acc_ref[...] += jnp.sum(hm, axis=0, keepdims=True)

    @pl.when(i == pl.num_programs(0) - 1)
    def _():
        v_ref[...] = (jnp.dot(acc_ref[...] * (1.0 / N), Wv_ref[...],
                              preferred_element_type=jnp.float32) + bv_ref[...])


def _tc_final(oe, hp, b2, Wp, bp2, Wv, bv2):
    return pl.pallas_call(
        _final_body,
        grid=(N // _BLK,),
        in_specs=[
            pl.BlockSpec((_BLK, EXT), lambda i: (i, 0)),
            pl.BlockSpec((_BLK, F), lambda i: (i, 0)),
            pl.BlockSpec((1, F), lambda i: (0, 0)),
            pl.BlockSpec((DH, 1), lambda i: (0, 0)),
            pl.BlockSpec((1, 1), lambda i: (0, 0)),
            pl.BlockSpec((DH, 1), lambda i: (0, 0)),
            pl.BlockSpec((1, 1), lambda i: (0, 0)),
        ],
        out_specs=[
            pl.BlockSpec((_BLK, 1), lambda i: (i, 0)),
            pl.BlockSpec((1, 1), lambda i: (0, 0)),
        ],
        out_shape=[
            jax.ShapeDtypeStruct((N, 1), jnp.float32),
            jax.ShapeDtypeStruct((1, 1), jnp.float32),
        ],
        scratch_shapes=[pltpu.VMEM((1, DH), jnp.float32)],
    )(oe, hp, b2, Wp, bp2, Wv, bv2)


_MESH = plsc.VectorSubcoreMesh(core_axis_name="c", subcore_axis_name="s")


@functools.partial(
    pl.kernel,
    mesh=_MESH,
    out_type=jax.ShapeDtypeStruct((N, EXT), jnp.float32),
    scratch_types=[
        pltpu.VMEM_SHARED((N, F), jnp.float32),
        pltpu.VMEM_SHARED((OUTS_ROWS, EXT), jnp.float32),
        pltpu.VMEM((N, 2 * H), jnp.float32),
        pltpu.VMEM((CHUNK,), jnp.int32),
        pltpu.VMEM((CHUNK,), jnp.int32),
        pltpu.VMEM((CCAP,), jnp.int32),
        pltpu.VMEM((CCAP,), jnp.int32),
        pltpu.VMEM((CCAP // SB, SB), jnp.int32),
        pltpu.VMEM((SB * H,), jnp.float32),
        pltpu.VMEM((SB, F), jnp.float32),
        pltpu.VMEM((SB, EXT), jnp.float32),
        pltpu.VMEM((16, EXT), jnp.float32),
    ],
)
def _sc_gat(feat_hbm, elr_hbm, src_hbm, dst_hbm, out_hbm,
            featS, outS, elr_v, sraw, draw, srcc, dstcf, dstc2,
            exv, gbuf, sbuf, zbuf):
    c = lax.axis_index("c")
    t = lax.axis_index("s")
    base = c * HALF
    iota = lax.iota(jnp.int32, 16)
    zf = jnp.zeros((16,), jnp.float32)
    zi = jnp.zeros((16,), jnp.int32)

    def _zb(r, _):
        def _zc(k, _):
            zbuf[r, pl.ds(k * 16, 16)] = zf
            return 0
        return lax.fori_loop(0, EXT // 16, _zc, 0)
    lax.fori_loop(0, 16, _zb, 0)

    def _zo(i, _):
        pltpu.sync_copy(zbuf, outS.at[pl.ds(t * STRIPE + i * 16, 16)])
        return 0
    lax.fori_loop(0, STRIPE // 16, _zo, 0)

    def _fl(i, _):
        r0 = t * FROWS + i * 125
        pltpu.sync_copy(feat_hbm.at[pl.ds(r0, 125)], gbuf.at[pl.ds(0, 125)])
        pltpu.sync_copy(gbuf.at[pl.ds(0, 125)], featS.at[pl.ds(r0, 125)])
        return 0
    lax.fori_loop(0, FROWS // 125, _fl, 0)

    pltpu.sync_copy(elr_hbm, elr_v)
    plsc.subcore_barrier()

    def _chunk(k, _):
        off = t * EPT + k * CHUNK
        pltpu.sync_copy(src_hbm.at[pl.ds(off, CHUNK)], sraw)
        pltpu.sync_copy(dst_hbm.at[pl.ds(off, CHUNK)], draw)

        di = zi + DUMMY

        def _pf(i, _):
            srcc[pl.ds(i * 16, 16)] = zi
            dstcf[pl.ds(i * 16, 16)] = di
            return 0
        lax.fori_loop(0, CCAP // 16, _pf, 0)

        def _cp(g, cnt):
            sv = sraw[pl.ds(g * 16, 16)]
            dv = draw[pl.ds(g * 16, 16)]
            dl = dv - base
            m = (dl >= 0) & (dl < HALF)
            plsc.store_compressed(srcc.at[pl.ds(cnt, 16)], sv, m)
            plsc.store_compressed(dstcf.at[pl.ds(cnt, 16)], dl, m)
            return cnt + jnp.sum(m.astype(jnp.int32))
        cnt = lax.fori_loop(0, CHUNK // 16, _cp, jnp.int32(0))

        def _c2(i, _):
            dstc2[i // (SB // 16), pl.ds((i % (SB // 16)) * 16, 16)] = \
                dstcf[pl.ds(i * 16, 16)]
            return 0
        lax.fori_loop(0, CCAP // 16, _c2, 0)

        ntr = (cnt + SB - 1) // SB

        def _trip(ti, _):
            pltpu.sync_copy(featS.at[srcc.at[pl.ds(ti * SB, SB)]], gbuf)

            def _ex(g, _):
                e0 = ti * SB + g * 4
                eidx = e0 + (iota >> 2)
                src16 = plsc.load_gather(srcc, [eidx])
                dst16 = plsc.load_gather(dstcf, [eidx])
                h16 = iota & 3
                elv = plsc.load_gather(elr_v, [src16, h16])
                erv = plsc.load_gather(elr_v, [dst16 + base, h16 + H])
                e = elv + erv
                e = jnp.where(e > 0, e, NEG * e)
                exv[pl.ds(g * 16, 16)] = jnp.exp(e)
                return 0
            lax.fori_loop(0, SB // 4, _ex, 0)

            def _row(r, _):
                for j in range(F // 16):
                    a = plsc.load_gather(exv, [zi + (r * H + j // 2)])
                    sbuf[r, pl.ds(j * 16, 16)] = a * gbuf[r, pl.ds(j * 16, 16)]
                exr = plsc.load_gather(exv, [r * H + (iota & 3)])
                sbuf[r, pl.ds(F, 16)] = jnp.where(iota < H, exr, 0.0)
                return 0
            lax.fori_loop(0, SB, _row, 0)

            pltpu.sync_copy(sbuf, outS.at[dstc2.at[ti]], add=True)
            return 0
        lax.fori_loop(0, ntr, _trip, 0)
        return 0
    lax.fori_loop(0, NCHUNK, _chunk, 0)

    plsc.subcore_barrier()

    @pl.when(t < NT - 1)
    def _():
        pltpu.sync_copy(outS.at[pl.ds(t * OROWS, OROWS)],
                        out_hbm.at[pl.ds(base + t * OROWS, OROWS)])

    @pl.when(t == NT - 1)
    def _():
        pltpu.sync_copy(outS.at[pl.ds((NT - 1) * OROWS, HALF - (NT - 1) * OROWS)],
                        out_hbm.at[pl.ds(base + (NT - 1) * OROWS,
                                         HALF - (NT - 1) * OROWS)])


def _blockdiag(al):
    heads = jnp.repeat(jnp.arange(H), DH)
    return jnp.where(heads[:, None] == jnp.arange(H)[None, :],
                     al.reshape(F, 1).astype(jnp.float32), 0.0)


def kernel(x, edge_index, W0, al0, ar0, b0, W1, al1, ar1, b1, Wp, bp, Wv, bv):
    src = edge_index[0].astype(jnp.int32)
    dst = edge_index[1].astype(jnp.int32)
    x = x.astype(jnp.float32)
    Al0, Ar0 = _blockdiag(al0), _blockdiag(ar0)
    Al1, Ar1 = _blockdiag(al1), _blockdiag(ar1)
    b0_2 = b0.astype(jnp.float32).reshape(1, F)
    b1_2 = b1.astype(jnp.float32).reshape(1, F)
    bp_2 = bp.astype(jnp.float32).reshape(1, 1)
    bv_2 = bv.astype(jnp.float32).reshape(1, 1)

    feat0, elr0 = _tc_feat(x, W0.astype(jnp.float32), Al0, Ar0)
    oe0 = _sc_gat(feat0, elr0, src, dst)
    h1, feat1, elr1 = _tc_post_feat(oe0, x, b0_2, W1.astype(jnp.float32),
                                    Al1, Ar1, residual=False, act=True)
    oe1 = _sc_gat(feat1, elr1, src, dst)
    h2, feat2, elr2 = _tc_post_feat(oe1, h1, b1_2, W1.astype(jnp.float32),
                                    Al1, Ar1, residual=True, act=False)
    oe2 = _sc_gat(feat2, elr2, src, dst)
    PI, V = _tc_final(oe2, h2, b1_2, Wp.astype(jnp.float32), bp_2,
                      Wv.astype(jnp.float32), bv_2)
    return (PI, V)


# speedup vs baseline: 29.8513x; 29.8513x over previous
import dataclasses
import functools

import jax
import jax.numpy as jnp
from jax import lax
from jax.experimental import pallas as pl
from jax.experimental.pallas import tpu as pltpu
from jax.experimental.pallas import tpu_sc as plsc

N = 10000
E = 640000
H = 4
DH = 32
F = H * DH
NEG = 0.2

HALF = N // 2
DUMMY = HALF
NORM_BASE = 5008
NROWS = 160
OUTF_ROWS = 5376
ELP_ROWS = 336
NT = 16
EPT = E // NT
CHUNK = 800
NCHUNK = EPT // CHUNK
SB = 64
CCAP = 1024
STRIPE = OUTF_ROWS // NT
OROWS = 312


_BLK = 2000


def _feat_body(h_ref, W_ref, Al_ref, Ar_ref, feat_ref, el_ref, er_ref):
    feat = jnp.dot(h_ref[...], W_ref[...], preferred_element_type=jnp.float32)
    feat_ref[...] = feat
    el_ref[...] = jnp.dot(feat, Al_ref[...], preferred_element_type=jnp.float32)
    er_ref[...] = jnp.dot(feat, Ar_ref[...], preferred_element_type=jnp.float32)


def _tc_feat(h, W, Al, Ar):
    return pl.pallas_call(
        _feat_body,
        grid=(N // _BLK,),
        in_specs=[
            pl.BlockSpec((_BLK, F), lambda i: (i, 0)),
            pl.BlockSpec((F, F), lambda i: (0, 0)),
            pl.BlockSpec((F, H), lambda i: (0, 0)),
            pl.BlockSpec((F, H), lambda i: (0, 0)),
        ],
        out_specs=[
            pl.BlockSpec((_BLK, F), lambda i: (i, 0)),
            pl.BlockSpec((_BLK, H), lambda i: (i, 0)),
            pl.BlockSpec((_BLK, H), lambda i: (i, 0)),
        ],
        out_shape=[
            jax.ShapeDtypeStruct((N, F), jnp.float32),
            jax.ShapeDtypeStruct((N, H), jnp.float32),
            jax.ShapeDtypeStruct((N, H), jnp.float32),
        ],
    )(h, W, Al, Ar)


def _normalize(agg, s):
    cols = []
    for hh in range(H):
        cols.append(agg[:, hh * DH:(hh + 1) * DH] / (s[:, hh:hh + 1] + 1e-9))
    return jnp.concatenate(cols, axis=1)


def _post_feat_body(residual, act, agg_ref, s_ref, hp_ref, b_ref, W_ref,
                    Al_ref, Ar_ref, h_ref, feat_ref, el_ref, er_ref):
    rst = _normalize(agg_ref[...], s_ref[...])
    if residual:
        rst = rst + hp_ref[...]
    rst = rst + b_ref[...]
    if act:
        rst = jnp.where(rst > 0, rst, jnp.exp(jnp.minimum(rst, 0.0)) - 1.0)
    h_ref[...] = rst
    feat = jnp.dot(rst, W_ref[...], preferred_element_type=jnp.float32)
    feat_ref[...] = feat
    el_ref[...] = jnp.dot(feat, Al_ref[...], preferred_element_type=jnp.float32)
    er_ref[...] = jnp.dot(feat, Ar_ref[...], preferred_element_type=jnp.float32)


def _tc_post_feat(agg, s, hp, b2, W, Al, Ar, residual, act):
    return pl.pallas_call(
        functools.partial(_post_feat_body, residual, act),
        grid=(N // _BLK,),
        in_specs=[
            pl.BlockSpec((_BLK, F), lambda i: (i, 0)),
            pl.BlockSpec((_BLK, H), lambda i: (i, 0)),
            pl.BlockSpec((_BLK, F), lambda i: (i, 0)),
            pl.BlockSpec((1, F), lambda i: (0, 0)),
            pl.BlockSpec((F, F), lambda i: (0, 0)),
            pl.BlockSpec((F, H), lambda i: (0, 0)),
            pl.BlockSpec((F, H), lambda i: (0, 0)),
        ],
        out_specs=[
            pl.BlockSpec((_BLK, F), lambda i: (i, 0)),
            pl.BlockSpec((_BLK, F), lambda i: (i, 0)),
            pl.BlockSpec((_BLK, H), lambda i: (i, 0)),
            pl.BlockSpec((_BLK, H), lambda i: (i, 0)),
        ],
        out_shape=[
            jax.ShapeDtypeStruct((N, F), jnp.float32),
            jax.ShapeDtypeStruct((N, F), jnp.float32),
            jax.ShapeDtypeStruct((N, H), jnp.float32),
            jax.ShapeDtypeStruct((N, H), jnp.float32),
        ],
    )(agg, s, hp, b2, W, Al, Ar)


def _final_body(agg_ref, s_ref, hp_ref, b_ref, Wp_ref, bp_ref, Wv_ref, bv_ref,
                pi_ref, v_ref, acc_ref):
    i = pl.program_id(0)
    rst = _normalize(agg_ref[...], s_ref[...]) + hp_ref[...] + b_ref[...]
    hm = (rst[:, 0:DH] + rst[:, DH:2 * DH] + rst[:, 2 * DH:3 * DH]
          + rst[:, 3 * DH:4 * DH]) * 0.25
    pi_ref[...] = (jnp.dot(hm, Wp_ref[...], preferred_element_type=jnp.float32)
                   + bp_ref[...])

    @pl.when(i == 0)
    def _():
        acc_ref[...] = jnp.zeros_like(acc_ref)

    acc_ref[...] += jnp.sum(hm, axis=0, keepdims=True)

    @pl.when(i == pl.num_programs(0) - 1)
    def _():
        v_ref[...] = (jnp.dot(acc_ref[...] * (1.0 / N), Wv_ref[...],
                              preferred_element_type=jnp.float32) + bv_ref[...])


def _tc_final(agg, s, hp, b2, Wp, bp2, Wv, bv2):
    return pl.pallas_call(
        _final_body,
        grid=(N // _BLK,),
        in_specs=[
            pl.BlockSpec((_BLK, F), lambda i: (i, 0)),
            pl.BlockSpec((_BLK, H), lambda i: (i, 0)),
            pl.BlockSpec((_BLK, F), lambda i: (i, 0)),
            pl.BlockSpec((1, F), lambda i: (0, 0)),
            pl.BlockSpec((DH, 1), lambda i: (0, 0)),
            pl.BlockSpec((1, 1), lambda i: (0, 0)),
            pl.BlockSpec((DH, 1), lambda i: (0, 0)),
            pl.BlockSpec((1, 1), lambda i: (0, 0)),
        ],
        out_specs=[
            pl.BlockSpec((_BLK, 1), lambda i: (i, 0)),
            pl.BlockSpec((1, 1), lambda i: (0, 0)),
        ],
        out_shape=[
            jax.ShapeDtypeStruct((N, 1), jnp.float32),
            jax.ShapeDtypeStruct((1, 1), jnp.float32),
        ],
        scratch_shapes=[pltpu.VMEM((1, DH), jnp.float32)],
    )(agg, s, hp, b2, Wp, bp2, Wv, bv2)


_SC_CACHE = []


def _sc_decorate(body):
    def call(*args):
        if not _SC_CACHE:
            mesh = plsc.VectorSubcoreMesh(core_axis_name="c", subcore_axis_name="s")
            cp = pltpu.CompilerParams()
            if "needs_layout_passes" in pltpu.CompilerParams.__dataclass_fields__:
                cp = dataclasses.replace(cp, needs_layout_passes=False)
            _SC_CACHE.append(functools.partial(
                pl.kernel,
                mesh=mesh,
                out_type=[
                    jax.ShapeDtypeStruct((N, F), jnp.float32),
                    jax.ShapeDtypeStruct((2 * NROWS, F), jnp.float32),
                ],
                compiler_params=cp,
                scratch_types=[
                    pltpu.VMEM_SHARED((OUTF_ROWS, F), jnp.float32),
                    pltpu.VMEM_SHARED((ELP_ROWS, F), jnp.float32),
                    pltpu.VMEM((H * HALF,), jnp.float32),
                    pltpu.VMEM((CHUNK,), jnp.int32),
                    pltpu.VMEM((CHUNK,), jnp.int32),
                    pltpu.VMEM((CCAP,), jnp.int32),
                    pltpu.VMEM((CCAP,), jnp.int32),
                    pltpu.VMEM((CCAP,), jnp.int32),
                    pltpu.VMEM((CCAP // SB, SB), jnp.int32),
                    pltpu.VMEM((CCAP // SB, SB), jnp.int32),
                    pltpu.VMEM((SB * H,), jnp.float32),
                    pltpu.VMEM((SB, F), jnp.float32),
                    pltpu.VMEM((SB, F), jnp.float32),
                    pltpu.VMEM((SB, F), jnp.float32),
                    pltpu.VMEM((SB, F), jnp.float32),
                    pltpu.VMEM((16, F), jnp.float32),
                ],
            )(body))
        return _SC_CACHE[0](*args)
    return call


@_sc_decorate
def _sc_gat(feat_hbm, elp_hbm, erf_hbm, src_hbm, dst_hbm, agg_hbm, s_hbm,
            outF, elS, er_v, sraw, draw, srcc, dstcf, elri, dstc2, nrm2,
            exv, gbuf, ebuf, sbuf, xbuf, zbuf):
    c = lax.axis_index("c")
    t = lax.axis_index("s")
    base = c * HALF
    iota = lax.iota(jnp.int32, 16)
    zf = jnp.zeros((16,), jnp.float32)
    zi = jnp.zeros((16,), jnp.int32)

    def _zb(r, _):
        def _zc(k, _):
            zbuf[r, pl.ds(k * 16, 16)] = zf
            return 0
        return lax.fori_loop(0, F // 16, _zc, 0)
    lax.fori_loop(0, 16, _zb, 0)

    def _zo(i, _):
        pltpu.sync_copy(zbuf, outF.at[pl.ds(t * STRIPE + i * 16, 16)])
        return 0
    lax.fori_loop(0, STRIPE // 16, _zo, 0)

    def _fl(i, _):
        s = t + i * NT

        @pl.when(s < ELP_ROWS // 16)
        def _():
            r0 = s * 16
            pltpu.sync_copy(elp_hbm.at[pl.ds(r0, 16)], gbuf.at[pl.ds(0, 16)])
            pltpu.sync_copy(gbuf.at[pl.ds(0, 16)], elS.at[pl.ds(r0, 16)])
        return 0
    lax.fori_loop(0, (ELP_ROWS // 16 + NT - 1) // NT, _fl, 0)

    pltpu.sync_copy(erf_hbm.at[pl.ds(base * H, H * HALF)], er_v)
    plsc.subcore_barrier()

    def _chunk(k, _):
        off = t * EPT + k * CHUNK
        pltpu.sync_copy(src_hbm.at[pl.ds(off, CHUNK)], sraw)
        pltpu.sync_copy(dst_hbm.at[pl.ds(off, CHUNK)], draw)

        di = zi + DUMMY

        def _pf(i, _):
            srcc[pl.ds(i * 16, 16)] = zi
            dstcf[pl.ds(i * 16, 16)] = di
            elri[pl.ds(i * 16, 16)] = zi
            return 0
        lax.fori_loop(0, CCAP // 16, _pf, 0)

        def _cp(g, cnt):
            sv = sraw[pl.ds(g * 16, 16)]
            dv = draw[pl.ds(g * 16, 16)]
            dl = dv - base
            m = (dl >= 0) & (dl < HALF)
            mi = m.astype(jnp.int32)
            cs = plsc.cumsum(mi)
            pos = jnp.where(m, cnt + cs - 1, (CCAP - 16) + iota)
            plsc.store_scatter(srcc, [pos], sv)
            plsc.store_scatter(dstcf, [pos], dl)
            plsc.store_scatter(elri, [pos], sv >> 5)
            return cnt + jnp.sum(mi, dtype=jnp.int32)
        cnt = lax.fori_loop(0, CHUNK // 16, _cp, jnp.int32(0))

        ntr = (cnt + (SB - 1)) >> 6

        def _c2(i, _):
            dlv = dstcf[pl.ds(i * 16, 16)]
            dstc2[i >> 2, pl.ds((i & 3) * 16, 16)] = dlv
            nrm2[i >> 2, pl.ds((i & 3) * 16, 16)] = NORM_BASE + (dlv >> 5)
            return 0
        lax.fori_loop(0, ntr << 2, _c2, 0)

        def _trip(ti, _):
            pltpu.sync_copy(feat_hbm.at[srcc.at[pl.ds(ti * SB, SB)]], gbuf)
            pltpu.sync_copy(elS.at[elri.at[pl.ds(ti * SB, SB)]], ebuf)

            def _ex(g, _):
                e0 = ti * SB + g * 4
                eidx = e0 + (iota >> 2)
                row16 = (g * 4) + (iota >> 2)
                h16 = iota & 3
                src16 = plsc.load_gather(srcc, [eidx])
                dst16 = plsc.load_gather(dstcf, [eidx])
                elv = plsc.load_gather(ebuf, [row16, ((src16 & 31) << 2) + h16])
                dstk = jnp.minimum(dst16, HALF - 1)
                erv = plsc.load_gather(er_v, [(dstk << 2) + h16])
                e = elv + erv
                e = jnp.where(e > 0, e, NEG * e)
                exv[pl.ds(g * 16, 16)] = jnp.exp(e)
                return 0
            lax.fori_loop(0, SB // 4, _ex, 0)

            def _row(r, _):
                for j in range(F // 16):
                    a = plsc.load_gather(exv, [zi + (r * H + (j >> 1))])
                    sbuf[r, pl.ds(j * 16, 16)] = a * gbuf[r, pl.ds(j * 16, 16)]
                    xbuf[r, pl.ds(j * 16, 16)] = zf
                exr = plsc.load_gather(exv, [r * H + (iota & 3)])
                dlv = plsc.load_gather(dstcf, [zi + (ti * SB + r)])
                cols = (((dlv & 31) << 2) + iota) & 127
                vals = jnp.where(iota < H, exr, 0.0)
                plsc.store_scatter(xbuf, [zi + r, cols], vals)
                return 0
            lax.fori_loop(0, SB, _row, 0)

            pltpu.sync_copy(sbuf, outF.at[dstc2.at[ti]], add=True)
            pltpu.sync_copy(xbuf, outF.at[nrm2.at[ti]], add=True)
            return 0
        lax.fori_loop(0, ntr, _trip, 0)
        return 0
    lax.fori_loop(0, NCHUNK, _chunk, 0)

    plsc.subcore_barrier()

    @pl.when(t < NT - 1)
    def _():
        pltpu.sync_copy(outF.at[pl.ds(t * OROWS, OROWS)],
                        agg_hbm.at[pl.ds(base + t * OROWS, OROWS)])

    @pl.when(t == NT - 1)
    def _():
        last = HALF - (NT - 1) * OROWS
        pltpu.sync_copy(outF.at[pl.ds((NT - 1) * OROWS, last)],
                        agg_hbm.at[pl.ds(base + (NT - 1) * OROWS, last)])

    @pl.when(t < NROWS // 16)
    def _():
        pltpu.sync_copy(outF.at[pl.ds(NORM_BASE + t * 16, 16)],
                        s_hbm.at[pl.ds(c * NROWS + t * 16, 16)])


def _blockdiag(al):
    heads = jnp.repeat(jnp.arange(H), DH)
    return jnp.where(heads[:, None] == jnp.arange(H)[None, :],
                     al.reshape(F, 1).astype(jnp.float32), 0.0)


def _layer_sc(feat, el, er, src, dst):
    elp = jnp.concatenate(
        [el.reshape(N * H), jnp.zeros((ELP_ROWS * F - N * H,), jnp.float32)]
    ).reshape(ELP_ROWS, F)
    erf = er.reshape(N * H)
    agg, s_packed = _sc_gat(feat, elp, erf, src, dst)
    s = s_packed.reshape(2, NROWS * F)[:, :HALF * H].reshape(N, H)
    return agg, s


def kernel(x, edge_index, W0, al0, ar0, b0, W1, al1, ar1, b1, Wp, bp, Wv, bv):
    with jax.enable_x64(False):
        return _kernel32(x, edge_index, W0, al0, ar0, b0, W1, al1, ar1, b1,
                         Wp, bp, Wv, bv)


def _kernel32(x, edge_index, W0, al0, ar0, b0, W1, al1, ar1, b1, Wp, bp, Wv, bv):
    src = edge_index[0].astype(jnp.int32)
    dst = edge_index[1].astype(jnp.int32)
    x = x.astype(jnp.float32)
    Al0, Ar0 = _blockdiag(al0), _blockdiag(ar0)
    Al1, Ar1 = _blockdiag(al1), _blockdiag(ar1)
    b0_2 = b0.astype(jnp.float32).reshape(1, F)
    b1_2 = b1.astype(jnp.float32).reshape(1, F)
    bp_2 = bp.astype(jnp.float32).reshape(1, 1)
    bv_2 = bv.astype(jnp.float32).reshape(1, 1)

    feat0, el0, er0 = _tc_feat(x, W0.astype(jnp.float32), Al0, Ar0)
    agg0, s0 = _layer_sc(feat0, el0, er0, src, dst)
    h1, feat1, el1, er1 = _tc_post_feat(agg0, s0, x, b0_2, W1.astype(jnp.float32),
                                        Al1, Ar1, residual=False, act=True)
    agg1, s1 = _layer_sc(feat1, el1, er1, src, dst)
    h2, feat2, el2, er2 = _tc_post_feat(agg1, s1, h1, b1_2, W1.astype(jnp.float32),
                                        Al1, Ar1, residual=True, act=False)
    agg2, s2 = _layer_sc(feat2, el2, er2, src, dst)
    PI, V = _tc_final(agg2, s2, h2, b1_2, Wp.astype(jnp.float32), bp_2,
                      Wv.astype(jnp.float32), bv_2)
    return (PI, V)


# speedup vs baseline: 30.1856x; 1.0112x over previous
import dataclasses
import functools

import jax
import jax.numpy as jnp
from jax import lax
from jax.experimental import pallas as pl
from jax.experimental.pallas import tpu as pltpu
from jax.experimental.pallas import tpu_sc as plsc

N = 10000
E = 640000
H = 4
DH = 32
F = H * DH
NEG = 0.2

HALF = N // 2
DUMMY = HALF
NORM_BASE = 5008
NROWS = 160
OUTF_ROWS = 5376
ELP_ROWS = 336
NT = 16
EPT = E // NT
CHUNK = 800
NCHUNK = EPT // CHUNK
SB = 64
CCAP = 1024
STRIPE = OUTF_ROWS // NT
OROWS = 312


_BLK = 2000


def _feat_body(h_ref, W_ref, Al_ref, Ar_ref, feat_ref, el_ref, er_ref):
    feat = jnp.dot(h_ref[...], W_ref[...], preferred_element_type=jnp.float32)
    feat_ref[...] = feat
    el_ref[...] = jnp.dot(feat, Al_ref[...], preferred_element_type=jnp.float32)
    er_ref[...] = jnp.dot(feat, Ar_ref[...], preferred_element_type=jnp.float32)


def _tc_feat(h, W, Al, Ar):
    return pl.pallas_call(
        _feat_body,
        grid=(N // _BLK,),
        in_specs=[
            pl.BlockSpec((_BLK, F), lambda i: (i, 0)),
            pl.BlockSpec((F, F), lambda i: (0, 0)),
            pl.BlockSpec((F, H), lambda i: (0, 0)),
            pl.BlockSpec((F, H), lambda i: (0, 0)),
        ],
        out_specs=[
            pl.BlockSpec((_BLK, F), lambda i: (i, 0)),
            pl.BlockSpec((_BLK, H), lambda i: (i, 0)),
            pl.BlockSpec((_BLK, H), lambda i: (i, 0)),
        ],
        out_shape=[
            jax.ShapeDtypeStruct((N, F), jnp.float32),
            jax.ShapeDtypeStruct((N, H), jnp.float32),
            jax.ShapeDtypeStruct((N, H), jnp.float32),
        ],
    )(h, W, Al, Ar)


def _normalize(agg, s):
    cols = []
    for hh in range(H):
        cols.append(agg[:, hh * DH:(hh + 1) * DH] / (s[:, hh:hh + 1] + 1e-9))
    return jnp.concatenate(cols, axis=1)


def _post_feat_body(residual, act, agg_ref, s_ref, hp_ref, b_ref, W_ref,
                    Al_ref, Ar_ref, h_ref, feat_ref, el_ref, er_ref):
    rst = _normalize(agg_ref[...], s_ref[...])
    if residual:
        rst = rst + hp_ref[...]
    rst = rst + b_ref[...]
    if act:
        rst = jnp.where(rst > 0, rst, jnp.exp(jnp.minimum(rst, 0.0)) - 1.0)
    h_ref[...] = rst
    feat = jnp.dot(rst, W_ref[...], preferred_element_type=jnp.float32)
    feat_ref[...] = feat
    el_ref[...] = jnp.dot(feat, Al_ref[...], preferred_element_type=jnp.float32)
    er_ref[...] = jnp.dot(feat, Ar_ref[...], preferred_element_type=jnp.float32)


def _tc_post_feat(agg, s, hp, b2, W, Al, Ar, residual, act):
    return pl.pallas_call(
        functools.partial(_post_feat_body, residual, act),
        grid=(N // _BLK,),
        in_specs=[
            pl.BlockSpec((_BLK, F), lambda i: (i, 0)),
            pl.BlockSpec((_BLK, H), lambda i: (i, 0)),
            pl.BlockSpec((_BLK, F), lambda i: (i, 0)),
            pl.BlockSpec((1, F), lambda i: (0, 0)),
            pl.BlockSpec((F, F), lambda i: (0, 0)),
            pl.BlockSpec((F, H), lambda i: (0, 0)),
            pl.BlockSpec((F, H), lambda i: (0, 0)),
        ],
        out_specs=[
            pl.BlockSpec((_BLK, F), lambda i: (i, 0)),
            pl.BlockSpec((_BLK, F), lambda i: (i, 0)),
            pl.BlockSpec((_BLK, H), lambda i: (i, 0)),
            pl.BlockSpec((_BLK, H), lambda i: (i, 0)),
        ],
        out_shape=[
            jax.ShapeDtypeStruct((N, F), jnp.float32),
            jax.ShapeDtypeStruct((N, F), jnp.float32),
            jax.ShapeDtypeStruct((N, H), jnp.float32),
            jax.ShapeDtypeStruct((N, H), jnp.float32),
        ],
    )(agg, s, hp, b2, W, Al, Ar)


def _final_body(agg_ref, s_ref, hp_ref, b_ref, Wp_ref, bp_ref, Wv_ref, bv_ref,
                pi_ref, v_ref, acc_ref):
    i = pl.program_id(0)
    rst = _normalize(agg_ref[...], s_ref[...]) + hp_ref[...] + b_ref[...]
    hm = (rst[:, 0:DH] + rst[:, DH:2 * DH] + rst[:, 2 * DH:3 * DH]
          + rst[:, 3 * DH:4 * DH]) * 0.25
    pi_ref[...] = (jnp.dot(hm, Wp_ref[...], preferred_element_type=jnp.float32)
                   + bp_ref[...])

    @pl.when(i == 0)
    def _():
        acc_ref[...] = jnp.zeros_like(acc_ref)

    acc_ref[...] += jnp.sum(hm, axis=0, keepdims=True)

    @pl.when(i == pl.num_programs(0) - 1)
    def _():
        v_ref[...] = (jnp.dot(acc_ref[...] * (1.0 / N), Wv_ref[...],
                              preferred_element_type=jnp.float32) + bv_ref[...])


def _tc_final(agg, s, hp, b2, Wp, bp2, Wv, bv2):
    return pl.pallas_call(
        _final_body,
        grid=(N // _BLK,),
        in_specs=[
            pl.BlockSpec((_BLK, F), lambda i: (i, 0)),
            pl.BlockSpec((_BLK, H), lambda i: (i, 0)),
            pl.BlockSpec((_BLK, F), lambda i: (i, 0)),
            pl.BlockSpec((1, F), lambda i: (0, 0)),
            pl.BlockSpec((DH, 1), lambda i: (0, 0)),
            pl.BlockSpec((1, 1), lambda i: (0, 0)),
            pl.BlockSpec((DH, 1), lambda i: (0, 0)),
            pl.BlockSpec((1, 1), lambda i: (0, 0)),
        ],
        out_specs=[
            pl.BlockSpec((_BLK, 1), lambda i: (i, 0)),
            pl.BlockSpec((1, 1), lambda i: (0, 0)),
        ],
        out_shape=[
            jax.ShapeDtypeStruct((N, 1), jnp.float32),
            jax.ShapeDtypeStruct((1, 1), jnp.float32),
        ],
        scratch_shapes=[pltpu.VMEM((1, DH), jnp.float32)],
    )(agg, s, hp, b2, Wp, bp2, Wv, bv2)


_SC_CACHE = []


def _sc_decorate(body):
    def call(*args):
        if not _SC_CACHE:
            mesh = plsc.VectorSubcoreMesh(core_axis_name="c", subcore_axis_name="s")
            cp = pltpu.CompilerParams()
            if "needs_layout_passes" in pltpu.CompilerParams.__dataclass_fields__:
                cp = dataclasses.replace(cp, needs_layout_passes=False)
            _SC_CACHE.append(functools.partial(
                pl.kernel,
                mesh=mesh,
                out_type=[
                    jax.ShapeDtypeStruct((N, F), jnp.float32),
                    jax.ShapeDtypeStruct((2 * NROWS, F), jnp.float32),
                ],
                compiler_params=cp,
                scratch_types=[
                    pltpu.VMEM_SHARED((OUTF_ROWS, F), jnp.float32),
                    pltpu.VMEM_SHARED((ELP_ROWS, F), jnp.float32),
                    pltpu.VMEM((H * HALF,), jnp.float32),
                    pltpu.VMEM((CHUNK,), jnp.int32),
                    pltpu.VMEM((CHUNK,), jnp.int32),
                    pltpu.VMEM((CCAP,), jnp.int32),
                    pltpu.VMEM((CCAP,), jnp.int32),
                    pltpu.VMEM((CCAP,), jnp.int32),
                    pltpu.VMEM((16, 2 * SB), jnp.int32),
                    pltpu.VMEM((SB * H,), jnp.float32),
                    pltpu.VMEM((SB, F), jnp.float32),
                    pltpu.VMEM((SB, F), jnp.float32),
                    pltpu.VMEM((SB, F), jnp.float32),
                    pltpu.VMEM((SB, F), jnp.float32),
                    pltpu.VMEM((2 * SB, F), jnp.float32),
                    pltpu.VMEM((16, F), jnp.float32),
                    pltpu.SemaphoreType.DMA,
                    pltpu.SemaphoreType.DMA,
                    pltpu.SemaphoreType.DMA,
                    pltpu.SemaphoreType.DMA,
                ],
            )(body))
        return _SC_CACHE[0](*args)
    return call


@_sc_decorate
def _sc_gat(feat_hbm, elp_hbm, erf_hbm, src_hbm, dst_hbm, agg_hbm, s_hbm,
            outF, elS, er_v, sraw, draw, srcc, dstcf, elri, cidx2,
            exv, gbufA, gbufB, ebufA, ebufB, sbig, zbuf,
            semg0, semg1, seme0, seme1):
    c = lax.axis_index("c")
    t = lax.axis_index("s")
    base = c * HALF
    iota = lax.iota(jnp.int32, 16)
    zf = jnp.zeros((16,), jnp.float32)
    zi = jnp.zeros((16,), jnp.int32)

    def _zb(r, _):
        def _zc(k, _):
            zbuf[r, pl.ds(k * 16, 16)] = zf
            return 0
        return lax.fori_loop(0, F // 16, _zc, 0)
    lax.fori_loop(0, 16, _zb, 0)

    def _zx(r, _):
        def _zc(k, _):
            sbig[SB + r, pl.ds(k * 16, 16)] = zf
            return 0
        return lax.fori_loop(0, F // 16, _zc, 0)
    lax.fori_loop(0, SB, _zx, 0)

    def _zo(i, _):
        pltpu.sync_copy(zbuf, outF.at[pl.ds(t * STRIPE + i * 16, 16)])
        return 0
    lax.fori_loop(0, STRIPE // 16, _zo, 0)

    def _fl(i, _):
        s = t + i * NT

        @pl.when(s < ELP_ROWS // 16)
        def _():
            r0 = s * 16
            pltpu.sync_copy(elp_hbm.at[pl.ds(r0, 16)], gbufA.at[pl.ds(0, 16)])
            pltpu.sync_copy(gbufA.at[pl.ds(0, 16)], elS.at[pl.ds(r0, 16)])
        return 0
    lax.fori_loop(0, (ELP_ROWS // 16 + NT - 1) // NT, _fl, 0)

    pltpu.sync_copy(erf_hbm.at[pl.ds(base * H, H * HALF)], er_v)
    plsc.subcore_barrier()

    def _chunk(k, _):
        off = t * EPT + k * CHUNK
        pltpu.sync_copy(src_hbm.at[pl.ds(off, CHUNK)], sraw)
        pltpu.sync_copy(dst_hbm.at[pl.ds(off, CHUNK)], draw)

        di = zi + DUMMY

        def _pf(i, _):
            srcc[pl.ds(i * 16, 16)] = zi
            dstcf[pl.ds(i * 16, 16)] = di
            elri[pl.ds(i * 16, 16)] = zi
            return 0
        lax.fori_loop(0, CCAP // 16, _pf, 0)

        def _cp(g, cnt):
            sv = sraw[pl.ds(g * 16, 16)]
            dv = draw[pl.ds(g * 16, 16)]
            dl = dv - base
            m = (dl >= 0) & (dl < HALF)
            mi = m.astype(jnp.int32)
            cs = plsc.cumsum(mi)
            pos = jnp.where(m, cnt + cs - 1, (CCAP - 16) + iota)
            plsc.store_scatter(srcc, [pos], sv)
            plsc.store_scatter(dstcf, [pos], dl)
            plsc.store_scatter(elri, [pos], sv >> 5)
            return cnt + jnp.sum(mi, dtype=jnp.int32)
        cnt = lax.fori_loop(0, CHUNK // 16, _cp, jnp.int32(0))

        ntr = (cnt + (SB - 1)) >> 6

        def _c2(i, _):
            dlv = dstcf[pl.ds(i * 16, 16)]
            cidx2[i >> 2, pl.ds((i & 3) * 16, 16)] = dlv
            cidx2[i >> 2, pl.ds(SB + (i & 3) * 16, 16)] = NORM_BASE + (dlv >> 5)
            return 0
        lax.fori_loop(0, ntr << 2, _c2, 0)

        def _issue(ti, gb, eb, sg, se):
            pltpu.async_copy(feat_hbm.at[srcc.at[pl.ds(ti * SB, SB)]], gb, sg)
            pltpu.async_copy(elS.at[elri.at[pl.ds(ti * SB, SB)]], eb, se)

        def _wait(ti, gb, eb, sg, se):
            pltpu.make_async_copy(
                feat_hbm.at[srcc.at[pl.ds(ti * SB, SB)]], gb, sg).wait()
            pltpu.make_async_copy(
                elS.at[elri.at[pl.ds(ti * SB, SB)]], eb, se).wait()

        def _compute(ti, gb, eb):
            def _ex(g, _):
                eidx = ti * SB + g * 4 + (iota >> 2)
                row16 = (g * 4) + (iota >> 2)
                h16 = iota & 3
                src16 = plsc.load_gather(srcc, [eidx])
                dst16 = plsc.load_gather(dstcf, [eidx])
                elv = plsc.load_gather(eb, [row16, ((src16 & 31) << 2) + h16])
                dstk = jnp.minimum(dst16, HALF - 1)
                erv = plsc.load_gather(er_v, [(dstk << 2) + h16])
                e = elv + erv
                e = jnp.where(e > 0, e, NEG * e)
                exv[pl.ds(g * 16, 16)] = jnp.exp(e)
                return 0
            lax.fori_loop(0, SB // 4, _ex, 0)

            def _row(r, _):
                a0 = plsc.load_gather(exv, [zi + (r * H + 0)])
                a1 = plsc.load_gather(exv, [zi + (r * H + 1)])
                a2 = plsc.load_gather(exv, [zi + (r * H + 2)])
                a3 = plsc.load_gather(exv, [zi + (r * H + 3)])
                heads = (a0, a0, a1, a1, a2, a2, a3, a3)
                for j in range(F // 16):
                    sbig[r, pl.ds(j * 16, 16)] = \
                        heads[j] * gb[r, pl.ds(j * 16, 16)]
                exr = plsc.load_gather(exv, [r * H + (iota & 3)])
                dlv = plsc.load_gather(dstcf, [zi + (ti * SB + r)])
                cols = (((dlv & 31) << 2) + iota) & 127
                vals = jnp.where(iota < H, exr, 0.0)
                plsc.store_scatter(sbig, [zi + (SB + r), cols], vals)
                return 0
            lax.fori_loop(0, SB, _row, 0)

            pltpu.sync_copy(sbig, outF.at[cidx2.at[ti]], add=True)

            def _rx(r, _):
                dlv = plsc.load_gather(dstcf, [zi + (ti * SB + r)])
                cols = (((dlv & 31) << 2) + iota) & 127
                plsc.store_scatter(sbig, [zi + (SB + r), cols], zf)
                return 0
            lax.fori_loop(0, SB, _rx, 0)

        def _trip(ti, _):
            pltpu.sync_copy(feat_hbm.at[srcc.at[pl.ds(ti * SB, SB)]], gbufA)
            pltpu.sync_copy(elS.at[elri.at[pl.ds(ti * SB, SB)]], ebufA)
            _compute(ti, gbufA, ebufA)
            return 0
        lax.fori_loop(0, ntr, _trip, 0)
        return 0
    lax.fori_loop(0, NCHUNK, _chunk, 0)

    plsc.subcore_barrier()

    @pl.when(t < NT - 1)
    def _():
        pltpu.sync_copy(outF.at[pl.ds(t * OROWS, OROWS)],
                        agg_hbm.at[pl.ds(base + t * OROWS, OROWS)])

    @pl.when(t == NT - 1)
    def _():
        last = HALF - (NT - 1) * OROWS
        pltpu.sync_copy(outF.at[pl.ds((NT - 1) * OROWS, last)],
                        agg_hbm.at[pl.ds(base + (NT - 1) * OROWS, last)])

    @pl.when(t < NROWS // 16)
    def _():
        pltpu.sync_copy(outF.at[pl.ds(NORM_BASE + t * 16, 16)],
                        s_hbm.at[pl.ds(c * NROWS + t * 16, 16)])


def _blockdiag(al):
    heads = jnp.repeat(jnp.arange(H), DH)
    return jnp.where(heads[:, None] == jnp.arange(H)[None, :],
                     al.reshape(F, 1).astype(jnp.float32), 0.0)


def _layer_sc(feat, el, er, src, dst):
    elp = jnp.concatenate(
        [el.reshape(N * H), jnp.zeros((ELP_ROWS * F - N * H,), jnp.float32)]
    ).reshape(ELP_ROWS, F)
    erf = er.reshape(N * H)
    agg, s_packed = _sc_gat(feat, elp, erf, src, dst)
    s = s_packed.reshape(2, NROWS * F)[:, :HALF * H].reshape(N, H)
    return agg, s


def kernel(x, edge_index, W0, al0, ar0, b0, W1, al1, ar1, b1, Wp, bp, Wv, bv):
    with jax.enable_x64(False):
        return _kernel32(x, edge_index, W0, al0, ar0, b0, W1, al1, ar1, b1,
                         Wp, bp, Wv, bv)


def _kernel32(x, edge_index, W0, al0, ar0, b0, W1, al1, ar1, b1, Wp, bp, Wv, bv):
    src = edge_index[0].astype(jnp.int32)
    dst = edge_index[1].astype(jnp.int32)
    x = x.astype(jnp.float32)
    Al0, Ar0 = _blockdiag(al0), _blockdiag(ar0)
    Al1, Ar1 = _blockdiag(al1), _blockdiag(ar1)
    b0_2 = b0.astype(jnp.float32).reshape(1, F)
    b1_2 = b1.astype(jnp.float32).reshape(1, F)
    bp_2 = bp.astype(jnp.float32).reshape(1, 1)
    bv_2 = bv.astype(jnp.float32).reshape(1, 1)

    feat0, el0, er0 = _tc_feat(x, W0.astype(jnp.float32), Al0, Ar0)
    agg0, s0 = _layer_sc(feat0, el0, er0, src, dst)
    h1, feat1, el1, er1 = _tc_post_feat(agg0, s0, x, b0_2, W1.astype(jnp.float32),
                                        Al1, Ar1, residual=False, act=True)
    agg1, s1 = _layer_sc(feat1, el1, er1, src, dst)
    h2, feat2, el2, er2 = _tc_post_feat(agg1, s1, h1, b1_2, W1.astype(jnp.float32),
                                        Al1, Ar1, residual=True, act=False)
    agg2, s2 = _layer_sc(feat2, el2, er2, src, dst)
    PI, V = _tc_final(agg2, s2, h2, b1_2, Wp.astype(jnp.float32), bp_2,
                      Wv.astype(jnp.float32), bv_2)
    return (PI, V)


# speedup vs baseline: 30.3203x; 1.0045x over previous
import dataclasses
import functools

import jax
import jax.numpy as jnp
from jax import lax
from jax.experimental import pallas as pl
from jax.experimental.pallas import tpu as pltpu
from jax.experimental.pallas import tpu_sc as plsc

N = 10000
E = 640000
H = 4
DH = 32
F = H * DH
NEG = 0.2

HALF = N // 2
DUMMY = HALF
NORM_BASE = 5008
NROWS = 160
OUTF_ROWS = 5376
FW = 256
NT = 16
EPT = E // NT
CHUNK = 800
NCHUNK = EPT // CHUNK
SB = 64
CCAP = 1024
STRIPE = OUTF_ROWS // NT
OROWS = 312


_BLK = 2000


def _featx(feat, Al, Ar):
    el = jnp.dot(feat, Al, preferred_element_type=jnp.float32)
    pad = jnp.zeros((feat.shape[0], FW - F - H), jnp.float32)
    return jnp.concatenate([feat, el, pad], axis=1)


def _feat_body(h_ref, W_ref, Al_ref, Ar_ref, featx_ref, er_ref):
    feat = jnp.dot(h_ref[...], W_ref[...], preferred_element_type=jnp.float32)
    featx_ref[...] = _featx(feat, Al_ref[...], Ar_ref[...])
    er_ref[...] = jnp.dot(feat, Ar_ref[...], preferred_element_type=jnp.float32)


def _tc_feat(h, W, Al, Ar):
    return pl.pallas_call(
        _feat_body,
        grid=(N // _BLK,),
        in_specs=[
            pl.BlockSpec((_BLK, F), lambda i: (i, 0)),
            pl.BlockSpec((F, F), lambda i: (0, 0)),
            pl.BlockSpec((F, H), lambda i: (0, 0)),
            pl.BlockSpec((F, H), lambda i: (0, 0)),
        ],
        out_specs=[
            pl.BlockSpec((_BLK, FW), lambda i: (i, 0)),
            pl.BlockSpec((_BLK, H), lambda i: (i, 0)),
        ],
        out_shape=[
            jax.ShapeDtypeStruct((N, FW), jnp.float32),
            jax.ShapeDtypeStruct((N, H), jnp.float32),
        ],
    )(h, W, Al, Ar)


def _normalize(agg, s):
    cols = []
    for hh in range(H):
        cols.append(agg[:, hh * DH:(hh + 1) * DH] / (s[:, hh:hh + 1] + 1e-9))
    return jnp.concatenate(cols, axis=1)


def _post_feat_body(residual, act, agg_ref, s_ref, hp_ref, b_ref, W_ref,
                    Al_ref, Ar_ref, h_ref, featx_ref, er_ref):
    rst = _normalize(agg_ref[...], s_ref[...])
    if residual:
        rst = rst + hp_ref[...]
    rst = rst + b_ref[...]
    if act:
        rst = jnp.where(rst > 0, rst, jnp.exp(jnp.minimum(rst, 0.0)) - 1.0)
    h_ref[...] = rst
    feat = jnp.dot(rst, W_ref[...], preferred_element_type=jnp.float32)
    featx_ref[...] = _featx(feat, Al_ref[...], Ar_ref[...])
    er_ref[...] = jnp.dot(feat, Ar_ref[...], preferred_element_type=jnp.float32)


def _tc_post_feat(agg, s, hp, b2, W, Al, Ar, residual, act):
    return pl.pallas_call(
        functools.partial(_post_feat_body, residual, act),
        grid=(N // _BLK,),
        in_specs=[
            pl.BlockSpec((_BLK, F), lambda i: (i, 0)),
            pl.BlockSpec((_BLK, H), lambda i: (i, 0)),
            pl.BlockSpec((_BLK, F), lambda i: (i, 0)),
            pl.BlockSpec((1, F), lambda i: (0, 0)),
            pl.BlockSpec((F, F), lambda i: (0, 0)),
            pl.BlockSpec((F, H), lambda i: (0, 0)),
            pl.BlockSpec((F, H), lambda i: (0, 0)),
        ],
        out_specs=[
            pl.BlockSpec((_BLK, F), lambda i: (i, 0)),
            pl.BlockSpec((_BLK, FW), lambda i: (i, 0)),
            pl.BlockSpec((_BLK, H), lambda i: (i, 0)),
        ],
        out_shape=[
            jax.ShapeDtypeStruct((N, F), jnp.float32),
            jax.ShapeDtypeStruct((N, FW), jnp.float32),
            jax.ShapeDtypeStruct((N, H), jnp.float32),
        ],
    )(agg, s, hp, b2, W, Al, Ar)


def _final_body(agg_ref, s_ref, hp_ref, b_ref, Wp_ref, bp_ref, Wv_ref, bv_ref,
                pi_ref, v_ref, acc_ref):
    i = pl.program_id(0)
    rst = _normalize(agg_ref[...], s_ref[...]) + hp_ref[...] + b_ref[...]
    hm = (rst[:, 0:DH] + rst[:, DH:2 * DH] + rst[:, 2 * DH:3 * DH]
          + rst[:, 3 * DH:4 * DH]) * 0.25
    pi_ref[...] = (jnp.dot(hm, Wp_ref[...], preferred_element_type=jnp.float32)
                   + bp_ref[...])

    @pl.when(i == 0)
    def _():
        acc_ref[...] = jnp.zeros_like(acc_ref)

    acc_ref[...] += jnp.sum(hm, axis=0, keepdims=True)

    @pl.when(i == pl.num_programs(0) - 1)
    def _():
        v_ref[...] = (jnp.dot(acc_ref[...] * (1.0 / N), Wv_ref[...],
                              preferred_element_type=jnp.float32) + bv_ref[...])


def _tc_final(agg, s, hp, b2, Wp, bp2, Wv, bv2):
    return pl.pallas_call(
        _final_body,
        grid=(N // _BLK,),
        in_specs=[
            pl.BlockSpec((_BLK, F), lambda i: (i, 0)),
            pl.BlockSpec((_BLK, H), lambda i: (i, 0)),
            pl.BlockSpec((_BLK, F), lambda i: (i, 0)),
            pl.BlockSpec((1, F), lambda i: (0, 0)),
            pl.BlockSpec((DH, 1), lambda i: (0, 0)),
            pl.BlockSpec((1, 1), lambda i: (0, 0)),
            pl.BlockSpec((DH, 1), lambda i: (0, 0)),
            pl.BlockSpec((1, 1), lambda i: (0, 0)),
        ],
        out_specs=[
            pl.BlockSpec((_BLK, 1), lambda i: (i, 0)),
            pl.BlockSpec((1, 1), lambda i: (0, 0)),
        ],
        out_shape=[
            jax.ShapeDtypeStruct((N, 1), jnp.float32),
            jax.ShapeDtypeStruct((1, 1), jnp.float32),
        ],
        scratch_shapes=[pltpu.VMEM((1, DH), jnp.float32)],
    )(agg, s, hp, b2, Wp, bp2, Wv, bv2)


_SC_CACHE = []


def _sc_decorate(body):
    def call(*args):
        if not _SC_CACHE:
            mesh = plsc.VectorSubcoreMesh(core_axis_name="c", subcore_axis_name="s")
            cp = pltpu.CompilerParams()
            if "needs_layout_passes" in pltpu.CompilerParams.__dataclass_fields__:
                cp = dataclasses.replace(cp, needs_layout_passes=False)
            _SC_CACHE.append(functools.partial(
                pl.kernel,
                mesh=mesh,
                out_type=[
                    jax.ShapeDtypeStruct((N, F), jnp.float32),
                    jax.ShapeDtypeStruct((2 * NROWS, F), jnp.float32),
                ],
                compiler_params=cp,
                scratch_types=[
                    pltpu.VMEM_SHARED((OUTF_ROWS, F), jnp.float32),
                    pltpu.VMEM((H * HALF,), jnp.float32),
                    pltpu.VMEM((CHUNK,), jnp.int32),
                    pltpu.VMEM((CHUNK,), jnp.int32),
                    pltpu.VMEM((CCAP,), jnp.int32),
                    pltpu.VMEM((CCAP,), jnp.int32),
                    pltpu.VMEM((16, 2 * SB), jnp.int32),
                    pltpu.VMEM((SB * H,), jnp.float32),
                    pltpu.VMEM((SB, FW), jnp.float32),
                    pltpu.VMEM((SB, FW), jnp.float32),
                    pltpu.VMEM((2 * SB, F), jnp.float32),
                    pltpu.VMEM((16, F), jnp.float32),
                    pltpu.SemaphoreType.DMA,
                    pltpu.SemaphoreType.DMA,
                ],
            )(body))
        return _SC_CACHE[0](*args)
    return call


@_sc_decorate
def _sc_gat(featx_hbm, erf_hbm, src_hbm, dst_hbm, agg_hbm, s_hbm,
            outF, er_v, sraw, draw, srcc, dstcf, cidx2,
            exv, gbufA, gbufB, sbig, zbuf, semg0, semg1):
    c = lax.axis_index("c")
    t = lax.axis_index("s")
    base = c * HALF
    iota = lax.iota(jnp.int32, 16)
    zf = jnp.zeros((16,), jnp.float32)
    zi = jnp.zeros((16,), jnp.int32)

    def _zb(r, _):
        def _zc(k, _):
            zbuf[r, pl.ds(k * 16, 16)] = zf
            return 0
        return lax.fori_loop(0, F // 16, _zc, 0)
    lax.fori_loop(0, 16, _zb, 0)

    def _zx(r, _):
        def _zc(k, _):
            sbig[SB + r, pl.ds(k * 16, 16)] = zf
            return 0
        return lax.fori_loop(0, F // 16, _zc, 0)
    lax.fori_loop(0, SB, _zx, 0)

    def _zo(i, _):
        pltpu.sync_copy(zbuf, outF.at[pl.ds(t * STRIPE + i * 16, 16)])
        return 0
    lax.fori_loop(0, STRIPE // 16, _zo, 0)

    pltpu.sync_copy(erf_hbm.at[pl.ds(base * H, H * HALF)], er_v)
    plsc.subcore_barrier()

    def _chunk(k, _):
        off = t * EPT + k * CHUNK
        pltpu.sync_copy(src_hbm.at[pl.ds(off, CHUNK)], sraw)
        pltpu.sync_copy(dst_hbm.at[pl.ds(off, CHUNK)], draw)

        di = zi + DUMMY

        def _pf(i, _):
            srcc[pl.ds(i * 16, 16)] = zi
            dstcf[pl.ds(i * 16, 16)] = di
            return 0
        lax.fori_loop(0, CCAP // 16, _pf, 0)

        def _cp(g, cnt):
            sv = sraw[pl.ds(g * 16, 16)]
            dv = draw[pl.ds(g * 16, 16)]
            dl = dv - base
            m = (dl >= 0) & (dl < HALF)
            mi = m.astype(jnp.int32)
            cs = plsc.cumsum(mi)
            pos = jnp.where(m, cnt + cs - 1, (CCAP - 16) + iota)
            plsc.store_scatter(srcc, [pos], sv)
            plsc.store_scatter(dstcf, [pos], dl)
            return cnt + jnp.sum(mi, dtype=jnp.int32)
        cnt = lax.fori_loop(0, CHUNK // 16, _cp, jnp.int32(0))

        ntr = (cnt + (SB - 1)) >> 6

        def _c2(i, _):
            dlv = dstcf[pl.ds(i * 16, 16)]
            cidx2[i >> 2, pl.ds((i & 3) * 16, 16)] = dlv
            cidx2[i >> 2, pl.ds(SB + (i & 3) * 16, 16)] = NORM_BASE + (dlv >> 5)
            return 0
        lax.fori_loop(0, ntr << 2, _c2, 0)

        def _issue(ti, gb, eb, sg, se):
            pltpu.async_copy(feat_hbm.at[srcc.at[pl.ds(ti * SB, SB)]], gb, sg)
            pltpu.async_copy(elS.at[elri.at[pl.ds(ti * SB, SB)]], eb, se)

        def _wait(ti, gb, eb, sg, se):
            pltpu.make_async_copy(
                feat_hbm.at[srcc.at[pl.ds(ti * SB, SB)]], gb, sg).wait()
            pltpu.make_async_copy(
                elS.at[elri.at[pl.ds(ti * SB, SB)]], eb, se).wait()

        def _ex_block(ti, eb):
            def _ex(g, _):
                eidx = ti * SB + g * 4 + (iota >> 2)
                row16 = (g * 4) + (iota >> 2)
                h16 = iota & 3
                src16 = plsc.load_gather(srcc, [eidx])
                dst16 = plsc.load_gather(dstcf, [eidx])
                elv = plsc.load_gather(eb, [row16, ((src16 & 31) << 2) + h16])
                dstk = jnp.minimum(dst16, HALF - 1)
                erv = plsc.load_gather(er_v, [(dstk << 2) + h16])
                e = elv + erv
                e = jnp.where(e > 0, e, NEG * e)
                exv[pl.ds(g * 16, 16)] = jnp.exp(e)
                return 0
            lax.fori_loop(0, SB // 4, _ex, 0)

        def _rows_block(ti, gb):
            def _row(r, _):
                a0 = plsc.load_gather(exv, [zi + (r * H + 0)])
                a1 = plsc.load_gather(exv, [zi + (r * H + 1)])
                a2 = plsc.load_gather(exv, [zi + (r * H + 2)])
                a3 = plsc.load_gather(exv, [zi + (r * H + 3)])
                heads = (a0, a0, a1, a1, a2, a2, a3, a3)
                for j in range(F // 16):
                    sbig[r, pl.ds(j * 16, 16)] = \
                        heads[j] * gb[r, pl.ds(j * 16, 16)]
                exr = plsc.load_gather(exv, [r * H + (iota & 3)])
                dlv = plsc.load_gather(dstcf, [zi + (ti * SB + r)])
                cols = (((dlv & 31) << 2) + iota) & 127
                vals = jnp.where(iota < H, exr, 0.0)
                plsc.store_scatter(sbig, [zi + (SB + r), cols], vals)
                return 0
            lax.fori_loop(0, SB, _row, 0)

            pltpu.async_copy(sbig, outF.at[cidx2.at[ti]], semg1, add=True)

        def _scatter_drain(ti):
            pltpu.make_async_copy(sbig, outF.at[cidx2.at[ti]], semg1).wait()
            def _rx(r, _):
                dlv = plsc.load_gather(dstcf, [zi + (ti * SB + r)])
                cols = (((dlv & 31) << 2) + iota) & 127
                plsc.store_scatter(sbig, [zi + (SB + r), cols], zf)
                return 0
            lax.fori_loop(0, SB, _rx, 0)

        def _trip(ti, _):
            pltpu.async_copy(featx_hbm.at[srcc.at[pl.ds(ti * SB, SB)]],
                             gbufA, semg0)

            @pl.when(ti > 0)
            def _():
                _scatter_drain(ti - 1)
            pltpu.make_async_copy(featx_hbm.at[srcc.at[pl.ds(ti * SB, SB)]],
                                  gbufA, semg0).wait()
            _ex_block(ti, gbufA)
            _rows_block(ti, gbufA)
            return 0
        lax.fori_loop(0, ntr, _trip, 0)

        @pl.when(ntr > 0)
        def _():
            _scatter_drain(ntr - 1)
        return 0
    lax.fori_loop(0, NCHUNK, _chunk, 0)

    plsc.subcore_barrier()

    @pl.when(t < NT - 1)
    def _():
        pltpu.sync_copy(outF.at[pl.ds(t * OROWS, OROWS)],
                        agg_hbm.at[pl.ds(base + t * OROWS, OROWS)])

    @pl.when(t == NT - 1)
    def _():
        last = HALF - (NT - 1) * OROWS
        pltpu.sync_copy(outF.at[pl.ds((NT - 1) * OROWS, last)],
                        agg_hbm.at[pl.ds(base + (NT - 1) * OROWS, last)])

    @pl.when(t < NROWS // 16)
    def _():
        pltpu.sync_copy(outF.at[pl.ds(NORM_BASE + t * 16, 16)],
                        s_hbm.at[pl.ds(c * NROWS + t * 16, 16)])


def _blockdiag(al):
    heads = jnp.repeat(jnp.arange(H), DH)
    return jnp.where(heads[:, None] == jnp.arange(H)[None, :],
                     al.reshape(F, 1).astype(jnp.float32), 0.0)


def _layer_sc(featx, er, src, dst):
    erf = er.reshape(N * H)
    agg, s_packed = _sc_gat(featx, erf, src, dst)
    s = s_packed.reshape(2, NROWS * F)[:, :HALF * H].reshape(N, H)
    return agg, s


def kernel(x, edge_index, W0, al0, ar0, b0, W1, al1, ar1, b1, Wp, bp, Wv, bv):
    with jax.enable_x64(False):
        return _kernel32(x, edge_index, W0, al0, ar0, b0, W1, al1, ar1, b1,
                         Wp, bp, Wv, bv)


def _kernel32(x, edge_index, W0, al0, ar0, b0, W1, al1, ar1, b1, Wp, bp, Wv, bv):
    src = edge_index[0].astype(jnp.int32)
    dst = edge_index[1].astype(jnp.int32)
    x = x.astype(jnp.float32)
    Al0, Ar0 = _blockdiag(al0), _blockdiag(ar0)
    Al1, Ar1 = _blockdiag(al1), _blockdiag(ar1)
    b0_2 = b0.astype(jnp.float32).reshape(1, F)
    b1_2 = b1.astype(jnp.float32).reshape(1, F)
    bp_2 = bp.astype(jnp.float32).reshape(1, 1)
    bv_2 = bv.astype(jnp.float32).reshape(1, 1)

    featx0, er0 = _tc_feat(x, W0.astype(jnp.float32), Al0, Ar0)
    agg0, s0 = _layer_sc(featx0, er0, src, dst)
    h1, featx1, er1 = _tc_post_feat(agg0, s0, x, b0_2, W1.astype(jnp.float32),
                                    Al1, Ar1, residual=False, act=True)
    agg1, s1 = _layer_sc(featx1, er1, src, dst)
    h2, featx2, er2 = _tc_post_feat(agg1, s1, h1, b1_2, W1.astype(jnp.float32),
                                    Al1, Ar1, residual=True, act=False)
    agg2, s2 = _layer_sc(featx2, er2, src, dst)
    PI, V = _tc_final(agg2, s2, h2, b1_2, Wp.astype(jnp.float32), bp_2,
                      Wv.astype(jnp.float32), bv_2)
    return (PI, V)


# speedup vs baseline: 30.8596x; 1.0178x over previous
import dataclasses
import functools

import jax
import jax.numpy as jnp
from jax import lax
from jax.experimental import pallas as pl
from jax.experimental.pallas import tpu as pltpu
from jax.experimental.pallas import tpu_sc as plsc

N = 10000
E = 640000
H = 4
DH = 32
F = H * DH
NEG = 0.2

HALF = N // 2
DUMMY = HALF
NORM_BASE = 5008
NROWS = 160
OUTF_ROWS = 5376
FW = 256
NT = 16
EPT = E // NT
CHUNK = 800
NCHUNK = EPT // CHUNK
SB = 64
CCAP = 1024
STRIPE = OUTF_ROWS // NT
OROWS = 312


_BLK = 2000


def _featx(feat, Al, Ar):
    el = jnp.dot(feat, Al, preferred_element_type=jnp.float32)
    pad = jnp.zeros((feat.shape[0], FW - F - H), jnp.float32)
    return jnp.concatenate([feat, el, pad], axis=1)


def _feat_body(h_ref, W_ref, Al_ref, Ar_ref, featx_ref, er_ref):
    feat = jnp.dot(h_ref[...], W_ref[...], preferred_element_type=jnp.float32)
    featx_ref[...] = _featx(feat, Al_ref[...], Ar_ref[...])
    er_ref[...] = jnp.dot(feat, Ar_ref[...], preferred_element_type=jnp.float32)


def _tc_feat(h, W, Al, Ar):
    return pl.pallas_call(
        _feat_body,
        grid=(N // _BLK,),
        in_specs=[
            pl.BlockSpec((_BLK, F), lambda i: (i, 0)),
            pl.BlockSpec((F, F), lambda i: (0, 0)),
            pl.BlockSpec((F, H), lambda i: (0, 0)),
            pl.BlockSpec((F, H), lambda i: (0, 0)),
        ],
        out_specs=[
            pl.BlockSpec((_BLK, FW), lambda i: (i, 0)),
            pl.BlockSpec((_BLK, H), lambda i: (i, 0)),
        ],
        out_shape=[
            jax.ShapeDtypeStruct((N, FW), jnp.float32),
            jax.ShapeDtypeStruct((N, H), jnp.float32),
        ],
    )(h, W, Al, Ar)


def _normalize(agg, s):
    cols = []
    for hh in range(H):
        cols.append(agg[:, hh * DH:(hh + 1) * DH] / (s[:, hh:hh + 1] + 1e-9))
    return jnp.concatenate(cols, axis=1)


def _post_feat_body(residual, act, agg_ref, s_ref, hp_ref, b_ref, W_ref,
                    Al_ref, Ar_ref, h_ref, featx_ref, er_ref):
    rst = _normalize(agg_ref[...], s_ref[...])
    if residual:
        rst = rst + hp_ref[...]
    rst = rst + b_ref[...]
    if act:
        rst = jnp.where(rst > 0, rst, jnp.exp(jnp.minimum(rst, 0.0)) - 1.0)
    h_ref[...] = rst
    feat = jnp.dot(rst, W_ref[...], preferred_element_type=jnp.float32)
    featx_ref[...] = _featx(feat, Al_ref[...], Ar_ref[...])
    er_ref[...] = jnp.dot(feat, Ar_ref[...], preferred_element_type=jnp.float32)


def _tc_post_feat(agg, s, hp, b2, W, Al, Ar, residual, act):
    return pl.pallas_call(
        functools.partial(_post_feat_body, residual, act),
        grid=(N // _BLK,),
        in_specs=[
            pl.BlockSpec((_BLK, F), lambda i: (i, 0)),
            pl.BlockSpec((_BLK, H), lambda i: (i, 0)),
            pl.BlockSpec((_BLK, F), lambda i: (i, 0)),
            pl.BlockSpec((1, F), lambda i: (0, 0)),
            pl.BlockSpec((F, F), lambda i: (0, 0)),
            pl.BlockSpec((F, H), lambda i: (0, 0)),
            pl.BlockSpec((F, H), lambda i: (0, 0)),
        ],
        out_specs=[
            pl.BlockSpec((_BLK, F), lambda i: (i, 0)),
            pl.BlockSpec((_BLK, FW), lambda i: (i, 0)),
            pl.BlockSpec((_BLK, H), lambda i: (i, 0)),
        ],
        out_shape=[
            jax.ShapeDtypeStruct((N, F), jnp.float32),
            jax.ShapeDtypeStruct((N, FW), jnp.float32),
            jax.ShapeDtypeStruct((N, H), jnp.float32),
        ],
    )(agg, s, hp, b2, W, Al, Ar)


def _final_body(agg_ref, s_ref, hp_ref, b_ref, Wp_ref, bp_ref, Wv_ref, bv_ref,
                pi_ref, v_ref, acc_ref):
    i = pl.program_id(0)
    rst = _normalize(agg_ref[...], s_ref[...]) + hp_ref[...] + b_ref[...]
    hm = (rst[:, 0:DH] + rst[:, DH:2 * DH] + rst[:, 2 * DH:3 * DH]
          + rst[:, 3 * DH:4 * DH]) * 0.25
    pi_ref[...] = (jnp.dot(hm, Wp_ref[...], preferred_element_type=jnp.float32)
                   + bp_ref[...])

    @pl.when(i == 0)
    def _():
        acc_ref[...] = jnp.zeros_like(acc_ref)

    acc_ref[...] += jnp.sum(hm, axis=0, keepdims=True)

    @pl.when(i == pl.num_programs(0) - 1)
    def _():
        v_ref[...] = (jnp.dot(acc_ref[...] * (1.0 / N), Wv_ref[...],
                              preferred_element_type=jnp.float32) + bv_ref[...])


def _tc_final(agg, s, hp, b2, Wp, bp2, Wv, bv2):
    return pl.pallas_call(
        _final_body,
        grid=(N // _BLK,),
        in_specs=[
            pl.BlockSpec((_BLK, F), lambda i: (i, 0)),
            pl.BlockSpec((_BLK, H), lambda i: (i, 0)),
            pl.BlockSpec((_BLK, F), lambda i: (i, 0)),
            pl.BlockSpec((1, F), lambda i: (0, 0)),
            pl.BlockSpec((DH, 1), lambda i: (0, 0)),
            pl.BlockSpec((1, 1), lambda i: (0, 0)),
            pl.BlockSpec((DH, 1), lambda i: (0, 0)),
            pl.BlockSpec((1, 1), lambda i: (0, 0)),
        ],
        out_specs=[
            pl.BlockSpec((_BLK, 1), lambda i: (i, 0)),
            pl.BlockSpec((1, 1), lambda i: (0, 0)),
        ],
        out_shape=[
            jax.ShapeDtypeStruct((N, 1), jnp.float32),
            jax.ShapeDtypeStruct((1, 1), jnp.float32),
        ],
        scratch_shapes=[pltpu.VMEM((1, DH), jnp.float32)],
    )(agg, s, hp, b2, Wp, bp2, Wv, bv2)


_SC_CACHE = []


def _sc_decorate(body):
    def call(*args):
        if not _SC_CACHE:
            mesh = plsc.VectorSubcoreMesh(core_axis_name="c", subcore_axis_name="s")
            cp = pltpu.CompilerParams()
            if "needs_layout_passes" in pltpu.CompilerParams.__dataclass_fields__:
                cp = dataclasses.replace(cp, needs_layout_passes=False)
            _SC_CACHE.append(functools.partial(
                pl.kernel,
                mesh=mesh,
                out_type=[
                    jax.ShapeDtypeStruct((N, F), jnp.float32),
                    jax.ShapeDtypeStruct((2 * NROWS, F), jnp.float32),
                ],
                compiler_params=cp,
                scratch_types=[
                    pltpu.VMEM_SHARED((OUTF_ROWS, F), jnp.float32),
                    pltpu.VMEM((H * HALF,), jnp.float32),
                    pltpu.VMEM((CHUNK,), jnp.int32),
                    pltpu.VMEM((CHUNK,), jnp.int32),
                    pltpu.VMEM((CCAP,), jnp.int32),
                    pltpu.VMEM((CCAP,), jnp.int32),
                    pltpu.VMEM((16, 2 * SB), jnp.int32),
                    pltpu.VMEM((SB * H,), jnp.float32),
                    pltpu.VMEM((SB, FW), jnp.float32),
                    pltpu.VMEM((SB, FW), jnp.float32),
                    pltpu.VMEM((2 * SB, F), jnp.float32),
                    pltpu.VMEM((16, F), jnp.float32),
                    pltpu.SemaphoreType.DMA,
                    pltpu.SemaphoreType.DMA,
                ],
            )(body))
        return _SC_CACHE[0](*args)
    return call


@_sc_decorate
def _sc_gat(featx_hbm, erf_hbm, src_hbm, dst_hbm, agg_hbm, s_hbm,
            outF, er_v, sraw, draw, srcc, dstcf, cidx2,
            exv, gbufA, gbufB, sbig, zbuf, semg0, semg1):
    c = lax.axis_index("c")
    t = lax.axis_index("s")
    base = c * HALF
    iota = lax.iota(jnp.int32, 16)
    zf = jnp.zeros((16,), jnp.float32)
    zi = jnp.zeros((16,), jnp.int32)

    def _zb(r, _):
        def _zc(k, _):
            zbuf[r, pl.ds(k * 16, 16)] = zf
            return 0
        return lax.fori_loop(0, F // 16, _zc, 0)
    lax.fori_loop(0, 16, _zb, 0)

    def _zx(r, _):
        def _zc(k, _):
            sbig[SB + r, pl.ds(k * 16, 16)] = zf
            return 0
        return lax.fori_loop(0, F // 16, _zc, 0)
    lax.fori_loop(0, SB, _zx, 0)

    def _zo(i, _):
        pltpu.sync_copy(zbuf, outF.at[pl.ds(t * STRIPE + i * 16, 16)])
        return 0
    lax.fori_loop(0, STRIPE // 16, _zo, 0)

    pltpu.sync_copy(erf_hbm.at[pl.ds(base * H, H * HALF)], er_v)
    plsc.subcore_barrier()

    def _chunk(k, _):
        off = t * EPT + k * CHUNK
        pltpu.sync_copy(src_hbm.at[pl.ds(off, CHUNK)], sraw)
        pltpu.sync_copy(dst_hbm.at[pl.ds(off, CHUNK)], draw)

        di = zi + DUMMY

        def _pf(i, _):
            srcc[pl.ds(i * 16, 16)] = zi
            dstcf[pl.ds(i * 16, 16)] = di
            return 0
        lax.fori_loop(0, CCAP // 16, _pf, 0)

        def _cp(g, cnt):
            sv = sraw[pl.ds(g * 16, 16)]
            dv = draw[pl.ds(g * 16, 16)]
            dl = dv - base
            m = (dl >= 0) & (dl < HALF)
            mi = m.astype(jnp.int32)
            cs = plsc.cumsum(mi)
            pos = jnp.where(m, cnt + cs - 1, (CCAP - 16) + iota)
            plsc.store_scatter(srcc, [pos], sv)
            plsc.store_scatter(dstcf, [pos], dl)
            return cnt + jnp.sum(mi, dtype=jnp.int32)
        cnt = lax.fori_loop(0, CHUNK // 16, _cp, jnp.int32(0))

        ntr = (cnt + (SB - 1)) >> 6

        def _c2(i, _):
            dlv = dstcf[pl.ds(i * 16, 16)]
            cidx2[i >> 2, pl.ds((i & 3) * 16, 16)] = dlv
            cidx2[i >> 2, pl.ds(SB + (i & 3) * 16, 16)] = NORM_BASE + (dlv >> 5)
            return 0
        lax.fori_loop(0, ntr << 2, _c2, 0)

        def _issue(ti, gb, eb, sg, se):
            pltpu.async_copy(feat_hbm.at[srcc.at[pl.ds(ti * SB, SB)]], gb, sg)
            pltpu.async_copy(elS.at[elri.at[pl.ds(ti * SB, SB)]], eb, se)

        def _wait(ti, gb, eb, sg, se):
            pltpu.make_async_copy(
                feat_hbm.at[srcc.at[pl.ds(ti * SB, SB)]], gb, sg).wait()
            pltpu.make_async_copy(
                elS.at[elri.at[pl.ds(ti * SB, SB)]], eb, se).wait()

        def _ex_block(ti, eb):
            def _ex(g, _):
                eidx = ti * SB + g * 4 + (iota >> 2)
                row16 = (g * 4) + (iota >> 2)
                h16 = iota & 3
                src16 = plsc.load_gather(srcc, [eidx])
                dst16 = plsc.load_gather(dstcf, [eidx])
                elv = plsc.load_gather(eb, [row16, ((src16 & 31) << 2) + h16])
                dstk = jnp.minimum(dst16, HALF - 1)
                erv = plsc.load_gather(er_v, [(dstk << 2) + h16])
                e = elv + erv
                e = jnp.where(e > 0, e, NEG * e)
                exv[pl.ds(g * 16, 16)] = jnp.exp(e)
                return 0
            lax.fori_loop(0, SB // 4, _ex, 0)

        def _rows_block(ti, gb):
            def _row(r, _):
                a0 = plsc.load_gather(exv, [zi + (r * H + 0)])
                a1 = plsc.load_gather(exv, [zi + (r * H + 1)])
                a2 = plsc.load_gather(exv, [zi + (r * H + 2)])
                a3 = plsc.load_gather(exv, [zi + (r * H + 3)])
                heads = (a0, a0, a1, a1, a2, a2, a3, a3)
                for j in range(F // 16):
                    sbig[r, pl.ds(j * 16, 16)] = \
                        heads[j] * gb[r, pl.ds(j * 16, 16)]
                exr = plsc.load_gather(exv, [r * H + (iota & 3)])
                dlv = plsc.load_gather(dstcf, [zi + (ti * SB + r)])
                cols = (((dlv & 31) << 2) + iota) & 127
                vals = jnp.where(iota < H, exr, 0.0)
                plsc.store_scatter(sbig, [zi + (SB + r), cols], vals)
                return 0
            lax.fori_loop(0, SB, _row, 0)

            pltpu.async_copy(sbig, outF.at[cidx2.at[ti]], semg1, add=True)

        def _scatter_drain(ti):
            pltpu.make_async_copy(sbig, outF.at[cidx2.at[ti]], semg1).wait()
            def _rx(r, _):
                dlv = plsc.load_gather(dstcf, [zi + (ti * SB + r)])
                cols = (((dlv & 31) << 2) + iota) & 127
                plsc.store_scatter(sbig, [zi + (SB + r), cols], zf)
                return 0
            lax.fori_loop(0, SB, _rx, 0)

        def _gissue(ti, gb, sg):
            pltpu.async_copy(featx_hbm.at[srcc.at[pl.ds(ti * SB, SB)]], gb, sg)

        def _gwait(ti, gb, sg):
            pltpu.make_async_copy(featx_hbm.at[srcc.at[pl.ds(ti * SB, SB)]],
                                  gb, sg).wait()

        @pl.when(ntr > 0)
        def _():
            _gissue(0, gbufA, semg0)

        def _pair(tj, _):
            t0 = tj * 2
            t1 = t0 + 1
            _gwait(t0, gbufA, semg0)

            @pl.when(t1 < ntr)
            def _():
                _gissue(t1, gbufB, semg2)

            @pl.when(t0 > 0)
            def _():
                _scatter_drain(t0 - 1)
            _ex_block(t0, gbufA)
            _rows_block(t0, gbufA)

            @pl.when(t1 < ntr)
            def _():
                _gwait(t1, gbufB, semg2)

                @pl.when(t0 + 2 < ntr)
                def _():
                    _gissue(t0 + 2, gbufA, semg0)

                _scatter_drain(t0)
                _ex_block(t1, gbufB)
                _rows_block(t1, gbufB)
            return 0
        lax.fori_loop(0, (ntr + 1) >> 1, _pair, 0)

        @pl.when(ntr > 0)
        def _():
            _scatter_drain(ntr - 1)
        return 0
    lax.fori_loop(0, NCHUNK, _chunk, 0)

    plsc.subcore_barrier()

    @pl.when(t < NT - 1)
    def _():
        pltpu.sync_copy(outF.at[pl.ds(t * OROWS, OROWS)],
                        agg_hbm.at[pl.ds(base + t * OROWS, OROWS)])

    @pl.when(t == NT - 1)
    def _():
        last = HALF - (NT - 1) * OROWS
        pltpu.sync_copy(outF.at[pl.ds((NT - 1) * OROWS, last)],
                        agg_hbm.at[pl.ds(base + (NT - 1) * OROWS, last)])

    @pl.when(t < NROWS // 16)
    def _():
        pltpu.sync_copy(outF.at[pl.ds(NORM_BASE + t * 16, 16)],
                        s_hbm.at[pl.ds(c * NROWS + t * 16, 16)])


def _blockdiag(al):
    heads = jnp.repeat(jnp.arange(H), DH)
    return jnp.where(heads[:, None] == jnp.arange(H)[None, :],
                     al.reshape(F, 1).astype(jnp.float32), 0.0)


def _layer_sc(featx, er, src, dst):
    erf = er.reshape(N * H)
    agg, s_packed = _sc_gat(featx, erf, src, dst)
    s = s_packed.reshape(2, NROWS * F)[:, :HALF * H].reshape(N, H)
    return agg, s


def kernel(x, edge_index, W0, al0, ar0, b0, W1, al1, ar1, b1, Wp, bp, Wv, bv):
    with jax.enable_x64(False):
        return _kernel32(x, edge_index, W0, al0, ar0, b0, W1, al1, ar1, b1,
                         Wp, bp, Wv, bv)


def _kernel32(x, edge_index, W0, al0, ar0, b0, W1, al1, ar1, b1, Wp, bp, Wv, bv):
    src = edge_index[0].astype(jnp.int32)
    dst = edge_index[1].astype(jnp.int32)
    x = x.astype(jnp.float32)
    Al0, Ar0 = _blockdiag(al0), _blockdiag(ar0)
    Al1, Ar1 = _blockdiag(al1), _blockdiag(ar1)
    b0_2 = b0.astype(jnp.float32).reshape(1, F)
    b1_2 = b1.astype(jnp.float32).reshape(1, F)
    bp_2 = bp.astype(jnp.float32).reshape(1, 1)
    bv_2 = bv.astype(jnp.float32).reshape(1, 1)

    featx0, er0 = _tc_feat(x, W0.astype(jnp.float32), Al0, Ar0)
    agg0, s0 = _layer_sc(featx0, er0, src, dst)
    h1, featx1, er1 = _tc_post_feat(agg0, s0, x, b0_2, W1.astype(jnp.float32),
                                    Al1, Ar1, residual=False, act=True)
    agg1, s1 = _layer_sc(featx1, er1, src, dst)
    h2, featx2, er2 = _tc_post_feat(agg1, s1, h1, b1_2, W1.astype(jnp.float32),
                                    Al1, Ar1, residual=True, act=False)
    agg2, s2 = _layer_sc(featx2, er2, src, dst)
    PI, V = _tc_final(agg2, s2, h2, b1_2, Wp.astype(jnp.float32), bp_2,
                      Wv.astype(jnp.float32), bv_2)
    return (PI, V)


# speedup vs baseline: 32.3785x; 1.0492x over previous
import dataclasses
import functools

import jax
import jax.numpy as jnp
from jax import lax
from jax.experimental import pallas as pl
from jax.experimental.pallas import tpu as pltpu
from jax.experimental.pallas import tpu_sc as plsc

N = 10000
E = 640000
H = 4
DH = 32
F = H * DH
NEG = 0.2

HALF = N // 2
DUMMY = HALF
NORM_BASE = 5008
NROWS = 160
OUTF_ROWS = 5376
ELP_ROWS = 336
NT = 16
EPT = E // NT
CHUNK = 800
NCHUNK = EPT // CHUNK
SB = 64
CCAP = 1024
STRIPE = OUTF_ROWS // NT
OROWS = 312


_BLK = 2000


def _feat_body(h_ref, W_ref, Al_ref, Ar_ref, feat_ref, el_ref, er_ref):
    feat = jnp.dot(h_ref[...], W_ref[...], preferred_element_type=jnp.float32)
    feat_ref[...] = feat
    el_ref[...] = jnp.dot(feat, Al_ref[...], preferred_element_type=jnp.float32)
    er_ref[...] = jnp.dot(feat, Ar_ref[...], preferred_element_type=jnp.float32)


def _tc_feat(h, W, Al, Ar):
    return pl.pallas_call(
        _feat_body,
        grid=(N // _BLK,),
        in_specs=[
            pl.BlockSpec((_BLK, F), lambda i: (i, 0)),
            pl.BlockSpec((F, F), lambda i: (0, 0)),
            pl.BlockSpec((F, H), lambda i: (0, 0)),
            pl.BlockSpec((F, H), lambda i: (0, 0)),
        ],
        out_specs=[
            pl.BlockSpec((_BLK, F), lambda i: (i, 0)),
            pl.BlockSpec((_BLK, H), lambda i: (i, 0)),
            pl.BlockSpec((_BLK, H), lambda i: (i, 0)),
        ],
        out_shape=[
            jax.ShapeDtypeStruct((N, F), jnp.float32),
            jax.ShapeDtypeStruct((N, H), jnp.float32),
            jax.ShapeDtypeStruct((N, H), jnp.float32),
        ],
    )(h, W, Al, Ar)


def _normalize(agg, s):
    cols = []
    for hh in range(H):
        cols.append(agg[:, hh * DH:(hh + 1) * DH] / (s[:, hh:hh + 1] + 1e-9))
    return jnp.concatenate(cols, axis=1)


def _post_feat_body(residual, act, agg_ref, s_ref, hp_ref, b_ref, W_ref,
                    Al_ref, Ar_ref, h_ref, feat_ref, el_ref, er_ref):
    rst = _normalize(agg_ref[...], s_ref[...])
    if residual:
        rst = rst + hp_ref[...]
    rst = rst + b_ref[...]
    if act:
        rst = jnp.where(rst > 0, rst, jnp.exp(jnp.minimum(rst, 0.0)) - 1.0)
    h_ref[...] = rst
    feat = jnp.dot(rst, W_ref[...], preferred_element_type=jnp.float32)
    feat_ref[...] = feat
    el_ref[...] = jnp.dot(feat, Al_ref[...], preferred_element_type=jnp.float32)
    er_ref[...] = jnp.dot(feat, Ar_ref[...], preferred_element_type=jnp.float32)


def _tc_post_feat(agg, s, hp, b2, W, Al, Ar, residual, act):
    return pl.pallas_call(
        functools.partial(_post_feat_body, residual, act),
        grid=(N // _BLK,),
        in_specs=[
            pl.BlockSpec((_BLK, F), lambda i: (i, 0)),
            pl.BlockSpec((_BLK, H), lambda i: (i, 0)),
            pl.BlockSpec((_BLK, F), lambda i: (i, 0)),
            pl.BlockSpec((1, F), lambda i: (0, 0)),
            pl.BlockSpec((F, F), lambda i: (0, 0)),
            pl.BlockSpec((F, H), lambda i: (0, 0)),
            pl.BlockSpec((F, H), lambda i: (0, 0)),
        ],
        out_specs=[
            pl.BlockSpec((_BLK, F), lambda i: (i, 0)),
            pl.BlockSpec((_BLK, F), lambda i: (i, 0)),
            pl.BlockSpec((_BLK, H), lambda i: (i, 0)),
            pl.BlockSpec((_BLK, H), lambda i: (i, 0)),
        ],
        out_shape=[
            jax.ShapeDtypeStruct((N, F), jnp.float32),
            jax.ShapeDtypeStruct((N, F), jnp.float32),
            jax.ShapeDtypeStruct((N, H), jnp.float32),
            jax.ShapeDtypeStruct((N, H), jnp.float32),
        ],
    )(agg, s, hp, b2, W, Al, Ar)


def _final_body(agg_ref, s_ref, hp_ref, b_ref, Wp_ref, bp_ref, Wv_ref, bv_ref,
                pi_ref, v_ref, acc_ref):
    i = pl.program_id(0)
    rst = _normalize(agg_ref[...], s_ref[...]) + hp_ref[...] + b_ref[...]
    hm = (rst[:, 0:DH] + rst[:, DH:2 * DH] + rst[:, 2 * DH:3 * DH]
          + rst[:, 3 * DH:4 * DH]) * 0.25
    pi_ref[...] = (jnp.dot(hm, Wp_ref[...], preferred_element_type=jnp.float32)
                   + bp_ref[...])

    @pl.when(i == 0)
    def _():
        acc_ref[...] = jnp.zeros_like(acc_ref)

    acc_ref[...] += jnp.sum(hm, axis=0, keepdims=True)

    @pl.when(i == pl.num_programs(0) - 1)
    def _():
        v_ref[...] = (jnp.dot(acc_ref[...] * (1.0 / N), Wv_ref[...],
                              preferred_element_type=jnp.float32) + bv_ref[...])


def _tc_final(agg, s, hp, b2, Wp, bp2, Wv, bv2):
    return pl.pallas_call(
        _final_body,
        grid=(N // _BLK,),
        in_specs=[
            pl.BlockSpec((_BLK, F), lambda i: (i, 0)),
            pl.BlockSpec((_BLK, H), lambda i: (i, 0)),
            pl.BlockSpec((_BLK, F), lambda i: (i, 0)),
            pl.BlockSpec((1, F), lambda i: (0, 0)),
            pl.BlockSpec((DH, 1), lambda i: (0, 0)),
            pl.BlockSpec((1, 1), lambda i: (0, 0)),
            pl.BlockSpec((DH, 1), lambda i: (0, 0)),
            pl.BlockSpec((1, 1), lambda i: (0, 0)),
        ],
        out_specs=[
            pl.BlockSpec((_BLK, 1), lambda i: (i, 0)),
            pl.BlockSpec((1, 1), lambda i: (0, 0)),
        ],
        out_shape=[
            jax.ShapeDtypeStruct((N, 1), jnp.float32),
            jax.ShapeDtypeStruct((1, 1), jnp.float32),
        ],
        scratch_shapes=[pltpu.VMEM((1, DH), jnp.float32)],
    )(agg, s, hp, b2, Wp, bp2, Wv, bv2)


_SC_CACHE = []


def _sc_decorate(body):
    def call(*args):
        if not _SC_CACHE:
            mesh = plsc.VectorSubcoreMesh(core_axis_name="c", subcore_axis_name="s")
            cp = pltpu.CompilerParams()
            if "needs_layout_passes" in pltpu.CompilerParams.__dataclass_fields__:
                cp = dataclasses.replace(cp, needs_layout_passes=False)
            _SC_CACHE.append(functools.partial(
                pl.kernel,
                mesh=mesh,
                out_type=[
                    jax.ShapeDtypeStruct((N, F), jnp.float32),
                    jax.ShapeDtypeStruct((2 * NROWS, F), jnp.float32),
                ],
                compiler_params=cp,
                scratch_types=[
                    pltpu.VMEM_SHARED((OUTF_ROWS, F), jnp.float32),
                    pltpu.VMEM_SHARED((ELP_ROWS, F), jnp.float32),
                    pltpu.VMEM((H * HALF,), jnp.float32),
                    pltpu.VMEM((CHUNK,), jnp.int32),
                    pltpu.VMEM((CHUNK,), jnp.int32),
                    pltpu.VMEM((CCAP,), jnp.int32),
                    pltpu.VMEM((CCAP,), jnp.int32),
                    pltpu.VMEM((CCAP,), jnp.int32),
                    pltpu.VMEM((16, 2 * SB), jnp.int32),
                    pltpu.VMEM((SB * H,), jnp.float32),
                    pltpu.VMEM((SB, F), jnp.float32),
                    pltpu.VMEM((SB, F), jnp.float32),
                    pltpu.VMEM((SB, F), jnp.float32),
                    pltpu.VMEM((SB, F), jnp.float32),
                    pltpu.VMEM((2 * SB, F), jnp.float32),
                    pltpu.VMEM((16, F), jnp.float32),
                    pltpu.SemaphoreType.DMA,
                    pltpu.SemaphoreType.DMA,
                    pltpu.SemaphoreType.DMA,
                    pltpu.SemaphoreType.DMA,
                ],
            )(body))
        return _SC_CACHE[0](*args)
    return call


@_sc_decorate
def _sc_gat(feat_hbm, elp_hbm, erf_hbm, src_hbm, dst_hbm, agg_hbm, s_hbm,
            outF, elS, er_v, sraw, draw, srcc, dstcf, elri, cidx2,
            exv, gbufA, gbufB, ebufA, ebufB, sbig, zbuf,
            semg0, semg1, seme0, seme1):
    c = lax.axis_index("c")
    t = lax.axis_index("s")
    base = c * HALF
    iota = lax.iota(jnp.int32, 16)
    zf = jnp.zeros((16,), jnp.float32)
    zi = jnp.zeros((16,), jnp.int32)

    def _zb(r, _):
        def _zc(k, _):
            zbuf[r, pl.ds(k * 16, 16)] = zf
            return 0
        return lax.fori_loop(0, F // 16, _zc, 0)
    lax.fori_loop(0, 16, _zb, 0)

    def _zx(r, _):
        def _zc(k, _):
            sbig[SB + r, pl.ds(k * 16, 16)] = zf
            return 0
        return lax.fori_loop(0, F // 16, _zc, 0)
    lax.fori_loop(0, SB, _zx, 0)

    def _zo(i, _):
        pltpu.sync_copy(zbuf, outF.at[pl.ds(t * STRIPE + i * 16, 16)])
        return 0
    lax.fori_loop(0, STRIPE // 16, _zo, 0)

    def _fl(i, _):
        s = t + i * NT

        @pl.when(s < ELP_ROWS // 16)
        def _():
            r0 = s * 16
            pltpu.sync_copy(elp_hbm.at[pl.ds(r0, 16)], gbufA.at[pl.ds(0, 16)])
            pltpu.sync_copy(gbufA.at[pl.ds(0, 16)], elS.at[pl.ds(r0, 16)])
        return 0
    lax.fori_loop(0, (ELP_ROWS // 16 + NT - 1) // NT, _fl, 0)

    pltpu.sync_copy(erf_hbm.at[pl.ds(base * H, H * HALF)], er_v)
    plsc.subcore_barrier()

    def _chunk(k, _):
        off = t * EPT + k * CHUNK
        pltpu.sync_copy(src_hbm.at[pl.ds(off, CHUNK)], sraw)
        pltpu.sync_copy(dst_hbm.at[pl.ds(off, CHUNK)], draw)

        di = zi + DUMMY

        def _pf(i, _):
            srcc[pl.ds(i * 16, 16)] = zi
            dstcf[pl.ds(i * 16, 16)] = di
            elri[pl.ds(i * 16, 16)] = zi
            return 0
        lax.fori_loop(0, CCAP // 16, _pf, 0)

        def _cp(g, cnt):
            sv = sraw[pl.ds(g * 16, 16)]
            dv = draw[pl.ds(g * 16, 16)]
            dl = dv - base
            m = (dl >= 0) & (dl < HALF)
            mi = m.astype(jnp.int32)
            cs = plsc.cumsum(mi)
            pos = jnp.where(m, cnt + cs - 1, (CCAP - 16) + iota)
            plsc.store_scatter(srcc, [pos], sv)
            plsc.store_scatter(dstcf, [pos], dl)
            plsc.store_scatter(elri, [pos], sv >> 5)
            return cnt + jnp.sum(mi, dtype=jnp.int32)
        cnt = lax.fori_loop(0, CHUNK // 16, _cp, jnp.int32(0))

        ntr = (cnt + (SB - 1)) >> 6

        def _c2(i, _):
            dlv = dstcf[pl.ds(i * 16, 16)]
            cidx2[i >> 2, pl.ds((i & 3) * 16, 16)] = dlv
            cidx2[i >> 2, pl.ds(SB + (i & 3) * 16, 16)] = NORM_BASE + (dlv >> 5)
            return 0
        lax.fori_loop(0, ntr << 2, _c2, 0)

        def _issue(ti, gb, eb, sg, se):
            pltpu.async_copy(feat_hbm.at[srcc.at[pl.ds(ti * SB, SB)]], gb, sg)
            pltpu.async_copy(elS.at[elri.at[pl.ds(ti * SB, SB)]], eb, se)

        def _wait(ti, gb, eb, sg, se):
            pltpu.make_async_copy(
                feat_hbm.at[srcc.at[pl.ds(ti * SB, SB)]], gb, sg).wait()
            pltpu.make_async_copy(
                elS.at[elri.at[pl.ds(ti * SB, SB)]], eb, se).wait()

        def _ex_block(ti, eb):
            def _ex(g, _):
                eidx = ti * SB + g * 4 + (iota >> 2)
                row16 = (g * 4) + (iota >> 2)
                h16 = iota & 3
                src16 = plsc.load_gather(srcc, [eidx])
                dst16 = plsc.load_gather(dstcf, [eidx])
                elv = plsc.load_gather(eb, [row16, ((src16 & 31) << 2) + h16])
                dstk = jnp.minimum(dst16, HALF - 1)
                erv = plsc.load_gather(er_v, [(dstk << 2) + h16])
                e = elv + erv
                e = jnp.where(e > 0, e, NEG * e)
                exv[pl.ds(g * 16, 16)] = jnp.exp(e)
                return 0
            lax.fori_loop(0, SB // 4, _ex, 0)

        def _rows_block(ti, gb):
            def _row(r, _):
                a0 = plsc.load_gather(exv, [zi + (r * H + 0)])
                a1 = plsc.load_gather(exv, [zi + (r * H + 1)])
                a2 = plsc.load_gather(exv, [zi + (r * H + 2)])
                a3 = plsc.load_gather(exv, [zi + (r * H + 3)])
                heads = (a0, a0, a1, a1, a2, a2, a3, a3)
                for j in range(F // 16):
                    sbig[r, pl.ds(j * 16, 16)] = \
                        heads[j] * gb[r, pl.ds(j * 16, 16)]
                exr = plsc.load_gather(exv, [r * H + (iota & 3)])
                dlv = plsc.load_gather(dstcf, [zi + (ti * SB + r)])
                cols = (((dlv & 31) << 2) + iota) & 127
                vals = jnp.where(iota < H, exr, 0.0)
                plsc.store_scatter(sbig, [zi + (SB + r), cols], vals)
                return 0
            lax.fori_loop(0, SB, _row, 0)

            pltpu.async_copy(sbig, outF.at[cidx2.at[ti]], semg1, add=True)

        def _scatter_drain(ti):
            pltpu.make_async_copy(sbig, outF.at[cidx2.at[ti]], semg1).wait()

            def _rx(r, _):
                dlv = plsc.load_gather(dstcf, [zi + (ti * SB + r)])
                cols = (((dlv & 31) << 2) + iota) & 127
                plsc.store_scatter(sbig, [zi + (SB + r), cols], zf)
                return 0
            lax.fori_loop(0, SB, _rx, 0)

        def _trip(ti, _):
            pltpu.async_copy(feat_hbm.at[srcc.at[pl.ds(ti * SB, SB)]],
                             gbufA, semg0)
            pltpu.sync_copy(elS.at[elri.at[pl.ds(ti * SB, SB)]], ebufA)
            _ex_block(ti, ebufA)

            @pl.when(ti > 0)
            def _():
                _scatter_drain(ti - 1)
            pltpu.make_async_copy(feat_hbm.at[srcc.at[pl.ds(ti * SB, SB)]],
                                  gbufA, semg0).wait()
            _rows_block(ti, gbufA)
            return 0
        lax.fori_loop(0, ntr, _trip, 0)

        @pl.when(ntr > 0)
        def _():
            _scatter_drain(ntr - 1)
        return 0
    lax.fori_loop(0, NCHUNK, _chunk, 0)

    plsc.subcore_barrier()

    @pl.when(t < NT - 1)
    def _():
        pltpu.sync_copy(outF.at[pl.ds(t * OROWS, OROWS)],
                        agg_hbm.at[pl.ds(base + t * OROWS, OROWS)])

    @pl.when(t == NT - 1)
    def _():
        last = HALF - (NT - 1) * OROWS
        pltpu.sync_copy(outF.at[pl.ds((NT - 1) * OROWS, last)],
                        agg_hbm.at[pl.ds(base + (NT - 1) * OROWS, last)])

    @pl.when(t < NROWS // 16)
    def _():
        pltpu.sync_copy(outF.at[pl.ds(NORM_BASE + t * 16, 16)],
                        s_hbm.at[pl.ds(c * NROWS + t * 16, 16)])


def _blockdiag(al):
    heads = jnp.repeat(jnp.arange(H), DH)
    return jnp.where(heads[:, None] == jnp.arange(H)[None, :],
                     al.reshape(F, 1).astype(jnp.float32), 0.0)


def _layer_sc(feat, el, er, src, dst):
    elp = jnp.concatenate(
        [el.reshape(N * H), jnp.zeros((ELP_ROWS * F - N * H,), jnp.float32)]
    ).reshape(ELP_ROWS, F)
    erf = er.reshape(N * H)
    agg, s_packed = _sc_gat(feat, elp, erf, src, dst)
    s = s_packed.reshape(2, NROWS * F)[:, :HALF * H].reshape(N, H)
    return agg, s


def kernel(x, edge_index, W0, al0, ar0, b0, W1, al1, ar1, b1, Wp, bp, Wv, bv):
    with jax.enable_x64(False):
        return _kernel32(x, edge_index, W0, al0, ar0, b0, W1, al1, ar1, b1,
                         Wp, bp, Wv, bv)


def _kernel32(x, edge_index, W0, al0, ar0, b0, W1, al1, ar1, b1, Wp, bp, Wv, bv):
    src = edge_index[0].astype(jnp.int32)
    dst = edge_index[1].astype(jnp.int32)
    x = x.astype(jnp.float32)
    Al0, Ar0 = _blockdiag(al0), _blockdiag(ar0)
    Al1, Ar1 = _blockdiag(al1), _blockdiag(ar1)
    b0_2 = b0.astype(jnp.float32).reshape(1, F)
    b1_2 = b1.astype(jnp.float32).reshape(1, F)
    bp_2 = bp.astype(jnp.float32).reshape(1, 1)
    bv_2 = bv.astype(jnp.float32).reshape(1, 1)

    feat0, el0, er0 = _tc_feat(x, W0.astype(jnp.float32), Al0, Ar0)
    agg0, s0 = _layer_sc(feat0, el0, er0, src, dst)
    h1, feat1, el1, er1 = _tc_post_feat(agg0, s0, x, b0_2, W1.astype(jnp.float32),
                                        Al1, Ar1, residual=False, act=True)
    agg1, s1 = _layer_sc(feat1, el1, er1, src, dst)
    h2, feat2, el2, er2 = _tc_post_feat(agg1, s1, h1, b1_2, W1.astype(jnp.float32),
                                        Al1, Ar1, residual=True, act=False)
    agg2, s2 = _layer_sc(feat2, el2, er2, src, dst)
    PI, V = _tc_final(agg2, s2, h2, b1_2, Wp.astype(jnp.float32), bp_2,
                      Wv.astype(jnp.float32), bv_2)
    return (PI, V)


# speedup vs baseline: 32.5875x; 1.0065x over previous
import dataclasses
import functools

import jax
import jax.numpy as jnp
from jax import lax
from jax.experimental import pallas as pl
from jax.experimental.pallas import tpu as pltpu
from jax.experimental.pallas import tpu_sc as plsc

N = 10000
E = 640000
H = 4
DH = 32
F = H * DH
NEG = 0.2

HALF = N // 2
DUMMY = HALF
NORM_BASE = 5008
NROWS = 160
OUTF_ROWS = 5376
ELP_ROWS = 336
NT = 16
EPT = E // NT
CHUNK = 800
NCHUNK = EPT // CHUNK
SB = 64
CCAP = 1024
STRIPE = OUTF_ROWS // NT
OROWS = 312


_BLK = 2000


def _feat_body(h_ref, W_ref, Al_ref, Ar_ref, feat_ref, el_ref, er_ref):
    feat = jnp.dot(h_ref[...], W_ref[...], preferred_element_type=jnp.float32)
    feat_ref[...] = feat
    el_ref[...] = jnp.dot(feat, Al_ref[...], preferred_element_type=jnp.float32)
    er_ref[...] = jnp.dot(feat, Ar_ref[...], preferred_element_type=jnp.float32)


def _tc_feat(h, W, Al, Ar):
    return pl.pallas_call(
        _feat_body,
        grid=(N // _BLK,),
        in_specs=[
            pl.BlockSpec((_BLK, F), lambda i: (i, 0)),
            pl.BlockSpec((F, F), lambda i: (0, 0)),
            pl.BlockSpec((F, H), lambda i: (0, 0)),
            pl.BlockSpec((F, H), lambda i: (0, 0)),
        ],
        out_specs=[
            pl.BlockSpec((_BLK, F), lambda i: (i, 0)),
            pl.BlockSpec((_BLK, H), lambda i: (i, 0)),
            pl.BlockSpec((_BLK, H), lambda i: (i, 0)),
        ],
        out_shape=[
            jax.ShapeDtypeStruct((N, F), jnp.float32),
            jax.ShapeDtypeStruct((N, H), jnp.float32),
            jax.ShapeDtypeStruct((N, H), jnp.float32),
        ],
    )(h, W, Al, Ar)


def _normalize(agg, s):
    cols = []
    for hh in range(H):
        cols.append(agg[:, hh * DH:(hh + 1) * DH] / (s[:, hh:hh + 1] + 1e-9))
    return jnp.concatenate(cols, axis=1)


def _post_feat_body(residual, act, agg_ref, s_ref, hp_ref, b_ref, W_ref,
                    Al_ref, Ar_ref, h_ref, feat_ref, el_ref, er_ref):
    rst = _normalize(agg_ref[...], s_ref[...])
    if residual:
        rst = rst + hp_ref[...]
    rst = rst + b_ref[...]
    if act:
        rst = jnp.where(rst > 0, rst, jnp.exp(jnp.minimum(rst, 0.0)) - 1.0)
    h_ref[...] = rst
    feat = jnp.dot(rst, W_ref[...], preferred_element_type=jnp.float32)
    feat_ref[...] = feat
    el_ref[...] = jnp.dot(feat, Al_ref[...], preferred_element_type=jnp.float32)
    er_ref[...] = jnp.dot(feat, Ar_ref[...], preferred_element_type=jnp.float32)


def _tc_post_feat(agg, s, hp, b2, W, Al, Ar, residual, act):
    return pl.pallas_call(
        functools.partial(_post_feat_body, residual, act),
        grid=(N // _BLK,),
        in_specs=[
            pl.BlockSpec((_BLK, F), lambda i: (i, 0)),
            pl.BlockSpec((_BLK, H), lambda i: (i, 0)),
            pl.BlockSpec((_BLK, F), lambda i: (i, 0)),
            pl.BlockSpec((1, F), lambda i: (0, 0)),
            pl.BlockSpec((F, F), lambda i: (0, 0)),
            pl.BlockSpec((F, H), lambda i: (0, 0)),
            pl.BlockSpec((F, H), lambda i: (0, 0)),
        ],
        out_specs=[
            pl.BlockSpec((_BLK, F), lambda i: (i, 0)),
            pl.BlockSpec((_BLK, F), lambda i: (i, 0)),
            pl.BlockSpec((_BLK, H), lambda i: (i, 0)),
            pl.BlockSpec((_BLK, H), lambda i: (i, 0)),
        ],
        out_shape=[
            jax.ShapeDtypeStruct((N, F), jnp.float32),
            jax.ShapeDtypeStruct((N, F), jnp.float32),
            jax.ShapeDtypeStruct((N, H), jnp.float32),
            jax.ShapeDtypeStruct((N, H), jnp.float32),
        ],
    )(agg, s, hp, b2, W, Al, Ar)


def _final_body(agg_ref, s_ref, hp_ref, b_ref, Wp_ref, bp_ref, Wv_ref, bv_ref,
                pi_ref, v_ref, acc_ref):
    i = pl.program_id(0)
    rst = _normalize(agg_ref[...], s_ref[...]) + hp_ref[...] + b_ref[...]
    hm = (rst[:, 0:DH] + rst[:, DH:2 * DH] + rst[:, 2 * DH:3 * DH]
          + rst[:, 3 * DH:4 * DH]) * 0.25
    pi_ref[...] = (jnp.dot(hm, Wp_ref[...], preferred_element_type=jnp.float32)
                   + bp_ref[...])

    @pl.when(i == 0)
    def _():
        acc_ref[...] = jnp.zeros_like(acc_ref)

    acc_ref[...] += jnp.sum(hm, axis=0, keepdims=True)

    @pl.when(i == pl.num_programs(0) - 1)
    def _():
        v_ref[...] = (jnp.dot(acc_ref[...] * (1.0 / N), Wv_ref[...],
                              preferred_element_type=jnp.float32) + bv_ref[...])


def _tc_final(agg, s, hp, b2, Wp, bp2, Wv, bv2):
    return pl.pallas_call(
        _final_body,
        grid=(N // _BLK,),
        in_specs=[
            pl.BlockSpec((_BLK, F), lambda i: (i, 0)),
            pl.BlockSpec((_BLK, H), lambda i: (i, 0)),
            pl.BlockSpec((_BLK, F), lambda i: (i, 0)),
            pl.BlockSpec((1, F), lambda i: (0, 0)),
            pl.BlockSpec((DH, 1), lambda i: (0, 0)),
            pl.BlockSpec((1, 1), lambda i: (0, 0)),
            pl.BlockSpec((DH, 1), lambda i: (0, 0)),
            pl.BlockSpec((1, 1), lambda i: (0, 0)),
        ],
        out_specs=[
            pl.BlockSpec((_BLK, 1), lambda i: (i, 0)),
            pl.BlockSpec((1, 1), lambda i: (0, 0)),
        ],
        out_shape=[
            jax.ShapeDtypeStruct((N, 1), jnp.float32),
            jax.ShapeDtypeStruct((1, 1), jnp.float32),
        ],
        scratch_shapes=[pltpu.VMEM((1, DH), jnp.float32)],
    )(agg, s, hp, b2, Wp, bp2, Wv, bv2)


_SC_CACHE = []


def _sc_decorate(body):
    def call(*args):
        if not _SC_CACHE:
            mesh = plsc.VectorSubcoreMesh(core_axis_name="c", subcore_axis_name="s")
            cp = pltpu.CompilerParams()
            if "needs_layout_passes" in pltpu.CompilerParams.__dataclass_fields__:
                cp = dataclasses.replace(cp, needs_layout_passes=False)
            _SC_CACHE.append(functools.partial(
                pl.kernel,
                mesh=mesh,
                out_type=[
                    jax.ShapeDtypeStruct((N, F), jnp.float32),
                    jax.ShapeDtypeStruct((2 * NROWS, F), jnp.float32),
                ],
                compiler_params=cp,
                scratch_types=[
                    pltpu.VMEM_SHARED((OUTF_ROWS, F), jnp.float32),
                    pltpu.VMEM_SHARED((ELP_ROWS, F), jnp.float32),
                    pltpu.VMEM((H * HALF,), jnp.float32),
                    pltpu.VMEM((CHUNK,), jnp.int32),
                    pltpu.VMEM((CHUNK,), jnp.int32),
                    pltpu.VMEM((CCAP,), jnp.int32),
                    pltpu.VMEM((CCAP,), jnp.int32),
                    pltpu.VMEM((CCAP,), jnp.int32),
                    pltpu.VMEM((16, 2 * SB), jnp.int32),
                    pltpu.VMEM((SB * H,), jnp.float32),
                    pltpu.VMEM((SB, F), jnp.float32),
                    pltpu.VMEM((SB, F), jnp.float32),
                    pltpu.VMEM((SB, F), jnp.float32),
                    pltpu.VMEM((SB, F), jnp.float32),
                    pltpu.VMEM((2 * SB, F), jnp.float32),
                    pltpu.VMEM((16, F), jnp.float32),
                    pltpu.SemaphoreType.DMA,
                    pltpu.SemaphoreType.DMA,
                    pltpu.SemaphoreType.DMA,
                    pltpu.SemaphoreType.DMA,
                ],
            )(body))
        return _SC_CACHE[0](*args)
    return call


@_sc_decorate
def _sc_gat(feat_hbm, elp_hbm, erf_hbm, src_hbm, dst_hbm, agg_hbm, s_hbm,
            outF, elS, er_v, sraw, draw, srcc, dstcf, elri, cidx2,
            exv, gbufA, gbufB, ebufA, ebufB, sbig, zbuf,
            semg0, semg1, seme0, seme1):
    c = lax.axis_index("c")
    t = lax.axis_index("s")
    base = c * HALF
    iota = lax.iota(jnp.int32, 16)
    zf = jnp.zeros((16,), jnp.float32)
    zi = jnp.zeros((16,), jnp.int32)

    def _zb(r, _):
        def _zc(k, _):
            zbuf[r, pl.ds(k * 16, 16)] = zf
            return 0
        return lax.fori_loop(0, F // 16, _zc, 0)
    lax.fori_loop(0, 16, _zb, 0)

    def _zx(r, _):
        def _zc(k, _):
            sbig[SB + r, pl.ds(k * 16, 16)] = zf
            return 0
        return lax.fori_loop(0, F // 16, _zc, 0)
    lax.fori_loop(0, SB, _zx, 0)

    def _zo(i, _):
        pltpu.sync_copy(zbuf, outF.at[pl.ds(t * STRIPE + i * 16, 16)])
        return 0
    lax.fori_loop(0, STRIPE // 16, _zo, 0)

    def _fl(i, _):
        s = t + i * NT

        @pl.when(s < ELP_ROWS // 16)
        def _():
            r0 = s * 16
            pltpu.sync_copy(elp_hbm.at[pl.ds(r0, 16)], gbufA.at[pl.ds(0, 16)])
            pltpu.sync_copy(gbufA.at[pl.ds(0, 16)], elS.at[pl.ds(r0, 16)])
        return 0
    lax.fori_loop(0, (ELP_ROWS // 16 + NT - 1) // NT, _fl, 0)

    pltpu.sync_copy(erf_hbm.at[pl.ds(base * H, H * HALF)], er_v)
    plsc.subcore_barrier()

    def _chunk(k, _):
        off = t * EPT + k * CHUNK
        pltpu.sync_copy(src_hbm.at[pl.ds(off, CHUNK)], sraw)
        pltpu.sync_copy(dst_hbm.at[pl.ds(off, CHUNK)], draw)

        di = zi + DUMMY

        def _pf(i, _):
            srcc[pl.ds(i * 16, 16)] = zi
            dstcf[pl.ds(i * 16, 16)] = di
            elri[pl.ds(i * 16, 16)] = zi
            return 0
        lax.fori_loop(0, CCAP // 16, _pf, 0)

        def _cp(g, cnt):
            sv = sraw[pl.ds(g * 16, 16)]
            dv = draw[pl.ds(g * 16, 16)]
            dl = dv - base
            m = (dl >= 0) & (dl < HALF)
            mi = m.astype(jnp.int32)
            cs = plsc.cumsum(mi)
            pos = jnp.where(m, cnt + cs - 1, (CCAP - 16) + iota)
            plsc.store_scatter(srcc, [pos], sv)
            plsc.store_scatter(dstcf, [pos], dl)
            plsc.store_scatter(elri, [pos], sv >> 5)
            return cnt + jnp.sum(mi, dtype=jnp.int32)
        cnt = lax.fori_loop(0, CHUNK // 16, _cp, jnp.int32(0))

        ntr = (cnt + (SB - 1)) >> 6

        def _c2(i, _):
            dlv = dstcf[pl.ds(i * 16, 16)]
            cidx2[i >> 2, pl.ds((i & 3) * 16, 16)] = dlv
            cidx2[i >> 2, pl.ds(SB + (i & 3) * 16, 16)] = NORM_BASE + (dlv >> 5)
            return 0
        lax.fori_loop(0, ntr << 2, _c2, 0)

        def _issue(ti, gb, eb, sg, se):
            pltpu.async_copy(feat_hbm.at[srcc.at[pl.ds(ti * SB, SB)]], gb, sg)
            pltpu.async_copy(elS.at[elri.at[pl.ds(ti * SB, SB)]], eb, se)

        def _wait(ti, gb, eb, sg, se):
            pltpu.make_async_copy(
                feat_hbm.at[srcc.at[pl.ds(ti * SB, SB)]], gb, sg).wait()
            pltpu.make_async_copy(
                elS.at[elri.at[pl.ds(ti * SB, SB)]], eb, se).wait()

        def _ex_block(ti, eb):
            def _ex(g, _):
                eidx = ti * SB + g * 4 + (iota >> 2)
                row16 = (g * 4) + (iota >> 2)
                h16 = iota & 3
                src16 = plsc.load_gather(srcc, [eidx])
                dst16 = plsc.load_gather(dstcf, [eidx])
                elv = plsc.load_gather(eb, [row16, ((src16 & 31) << 2) + h16])
                dstk = jnp.minimum(dst16, HALF - 1)
                erv = plsc.load_gather(er_v, [(dstk << 2) + h16])
                e = elv + erv
                e = jnp.where(e > 0, e, NEG * e)
                exv[pl.ds(g * 16, 16)] = jnp.exp(e)
                return 0
            lax.fori_loop(0, SB // 4, _ex, 0)

        def _rows_block(ti, gb):
            def _row(r, _):
                a0 = plsc.load_gather(exv, [zi + (r * H + 0)])
                a1 = plsc.load_gather(exv, [zi + (r * H + 1)])
                a2 = plsc.load_gather(exv, [zi + (r * H + 2)])
                a3 = plsc.load_gather(exv, [zi + (r * H + 3)])
                heads = (a0, a0, a1, a1, a2, a2, a3, a3)
                for j in range(F // 16):
                    sbig[r, pl.ds(j * 16, 16)] = \
                        heads[j] * gb[r, pl.ds(j * 16, 16)]
                exr = plsc.load_gather(exv, [r * H + (iota & 3)])
                dlv = plsc.load_gather(dstcf, [zi + (ti * SB + r)])
                cols = (((dlv & 31) << 2) + iota) & 127
                vals = jnp.where(iota < H, exr, 0.0)
                plsc.store_scatter(sbig, [zi + (SB + r), cols], vals)
                return 0
            lax.fori_loop(0, SB, _row, 0)

            pltpu.async_copy(sbig, outF.at[cidx2.at[ti]], semg1, add=True)

        def _scatter_drain(ti):
            pltpu.make_async_copy(sbig, outF.at[cidx2.at[ti]], semg1).wait()

            def _rx(r, _):
                dlv = plsc.load_gather(dstcf, [zi + (ti * SB + r)])
                cols = (((dlv & 31) << 2) + iota) & 127
                plsc.store_scatter(sbig, [zi + (SB + r), cols], zf)
                return 0
            lax.fori_loop(0, SB, _rx, 0)

        def _fissue(ti, gb, sg):
            pltpu.async_copy(feat_hbm.at[srcc.at[pl.ds(ti * SB, SB)]], gb, sg)

        def _fwait(ti, gb, sg):
            pltpu.make_async_copy(feat_hbm.at[srcc.at[pl.ds(ti * SB, SB)]],
                                  gb, sg).wait()

        def _body(ti, gb):
            pltpu.sync_copy(elS.at[elri.at[pl.ds(ti * SB, SB)]], ebufA)
            _ex_block(ti, ebufA)

            @pl.when(ti > 0)
            def _():
                _scatter_drain(ti - 1)
            _rows_block(ti, gb)

        @pl.when(ntr > 0)
        def _():
            _fissue(0, gbufA, semg0)

        def _pair(tj, _):
            t0 = tj * 2
            t1 = t0 + 1
            _fwait(t0, gbufA, semg0)

            @pl.when(t1 < ntr)
            def _():
                _fissue(t1, gbufB, seme0)
            _body(t0, gbufA)

            @pl.when(t1 < ntr)
            def _():
                _fwait(t1, gbufB, seme0)

                @pl.when(t0 + 2 < ntr)
                def _():
                    _fissue(t0 + 2, gbufA, semg0)
                _body(t1, gbufB)
            return 0
        lax.fori_loop(0, (ntr + 1) >> 1, _pair, 0)

        @pl.when(ntr > 0)
        def _():
            _scatter_drain(ntr - 1)
        return 0
    lax.fori_loop(0, NCHUNK, _chunk, 0)

    plsc.subcore_barrier()

    @pl.when(t < NT - 1)
    def _():
        pltpu.sync_copy(outF.at[pl.ds(t * OROWS, OROWS)],
                        agg_hbm.at[pl.ds(base + t * OROWS, OROWS)])

    @pl.when(t == NT - 1)
    def _():
        last = HALF - (NT - 1) * OROWS
        pltpu.sync_copy(outF.at[pl.ds((NT - 1) * OROWS, last)],
                        agg_hbm.at[pl.ds(base + (NT - 1) * OROWS, last)])

    @pl.when(t < NROWS // 16)
    def _():
        pltpu.sync_copy(outF.at[pl.ds(NORM_BASE + t * 16, 16)],
                        s_hbm.at[pl.ds(c * NROWS + t * 16, 16)])


def _blockdiag(al):
    heads = jnp.repeat(jnp.arange(H), DH)
    return jnp.where(heads[:, None] == jnp.arange(H)[None, :],
                     al.reshape(F, 1).astype(jnp.float32), 0.0)


def _layer_sc(feat, el, er, src, dst):
    elp = jnp.concatenate(
        [el.reshape(N * H), jnp.zeros((ELP_ROWS * F - N * H,), jnp.float32)]
    ).reshape(ELP_ROWS, F)
    erf = er.reshape(N * H)
    agg, s_packed = _sc_gat(feat, elp, erf, src, dst)
    s = s_packed.reshape(2, NROWS * F)[:, :HALF * H].reshape(N, H)
    return agg, s


def kernel(x, edge_index, W0, al0, ar0, b0, W1, al1, ar1, b1, Wp, bp, Wv, bv):
    with jax.enable_x64(False):
        return _kernel32(x, edge_index, W0, al0, ar0, b0, W1, al1, ar1, b1,
                         Wp, bp, Wv, bv)


def _kernel32(x, edge_index, W0, al0, ar0, b0, W1, al1, ar1, b1, Wp, bp, Wv, bv):
    src = edge_index[0].astype(jnp.int32)
    dst = edge_index[1].astype(jnp.int32)
    x = x.astype(jnp.float32)
    Al0, Ar0 = _blockdiag(al0), _blockdiag(ar0)
    Al1, Ar1 = _blockdiag(al1), _blockdiag(ar1)
    b0_2 = b0.astype(jnp.float32).reshape(1, F)
    b1_2 = b1.astype(jnp.float32).reshape(1, F)
    bp_2 = bp.astype(jnp.float32).reshape(1, 1)
    bv_2 = bv.astype(jnp.float32).reshape(1, 1)

    feat0, el0, er0 = _tc_feat(x, W0.astype(jnp.float32), Al0, Ar0)
    agg0, s0 = _layer_sc(feat0, el0, er0, src, dst)
    h1, feat1, el1, er1 = _tc_post_feat(agg0, s0, x, b0_2, W1.astype(jnp.float32),
                                        Al1, Ar1, residual=False, act=True)
    agg1, s1 = _layer_sc(feat1, el1, er1, src, dst)
    h2, feat2, el2, er2 = _tc_post_feat(agg1, s1, h1, b1_2, W1.astype(jnp.float32),
                                        Al1, Ar1, residual=True, act=False)
    agg2, s2 = _layer_sc(feat2, el2, er2, src, dst)
    PI, V = _tc_final(agg2, s2, h2, b1_2, Wp.astype(jnp.float32), bp_2,
                      Wv.astype(jnp.float32), bv_2)
    return (PI, V)
